# baseline (speedup 1.0000x reference)
import functools

import jax
import jax.numpy as jnp
from jax.experimental import pallas as pl

N = 10000
D = 256
R = 8
NB = 8
E = 160000
EPS = 1e-5

BN = 1000


def _mix_kernel(comp_ref, basis_ref, w_ref):
    w_ref[...] = jax.lax.dot_general(
        comp_ref[...], basis_ref[...], (((1,), (0,)), ((), ())),
        preferred_element_type=jnp.float32)


def _mix(comp, basis):
    return pl.pallas_call(
        _mix_kernel,
        out_shape=jax.ShapeDtypeStruct((R, D * D), jnp.float32),
    )(comp, basis.reshape(NB, D * D)).reshape(R, D, D)


def _trans_kernel(h_ref, w_ref, root_ref, trans_ref, hr_ref):
    h = h_ref[...]
    t = jax.lax.dot_general(h, w_ref[...], (((1,), (1,)), ((), ())),
                            preferred_element_type=jnp.float32)
    trans_ref[...] = t.reshape(BN, R * D)
    hr_ref[...] = jnp.dot(h, root_ref[...], preferred_element_type=jnp.float32)


def _trans(h, w, root):
    grid = (N // BN,)
    return pl.pallas_call(
        _trans_kernel,
        grid=grid,
        in_specs=[
            pl.BlockSpec((BN, D), lambda i: (i, 0)),
            pl.BlockSpec((R, D, D), lambda i: (0, 0, 0)),
            pl.BlockSpec((D, D), lambda i: (0, 0)),
        ],
        out_specs=[
            pl.BlockSpec((BN, R * D), lambda i: (i, 0)),
            pl.BlockSpec((BN, D), lambda i: (i, 0)),
        ],
        out_shape=[
            jax.ShapeDtypeStruct((N, R * D), jnp.float32),
            jax.ShapeDtypeStruct((N, D), jnp.float32),
        ],
    )(h, w, root)


def _post_kernel(agg_ref, hr_ref, h_ref, bias_ref, lnw_ref, lnb_ref, out_ref):
    m = agg_ref[...] + hr_ref[...] + bias_ref[...]
    m = 0.5 * m * (1.0 + jax.lax.erf(m * (2.0 ** -0.5)))
    x = h_ref[...] + m
    mu = jnp.mean(x, axis=-1, keepdims=True)
    var = jnp.mean((x - mu) ** 2, axis=-1, keepdims=True)
    out_ref[...] = (x - mu) / jnp.sqrt(var + EPS) * lnw_ref[...] + lnb_ref[...]


def _post(agg, hr, h, bias, lnw, lnb):
    grid = (N // BN,)
    vec = lambda i: (0, 0)
    return pl.pallas_call(
        _post_kernel,
        grid=grid,
        in_specs=[
            pl.BlockSpec((BN, D), lambda i: (i, 0)),
            pl.BlockSpec((BN, D), lambda i: (i, 0)),
            pl.BlockSpec((BN, D), lambda i: (i, 0)),
            pl.BlockSpec((1, D), vec),
            pl.BlockSpec((1, D), vec),
            pl.BlockSpec((1, D), vec),
        ],
        out_specs=pl.BlockSpec((BN, D), lambda i: (i, 0)),
        out_shape=jax.ShapeDtypeStruct((N, D), jnp.float32),
    )(agg, hr, h, bias.reshape(1, D), lnw.reshape(1, D), lnb.reshape(1, D))


def kernel(x_flat, edge_index, edge_type, valid_mask_flat,
           basis_0, comp_0, root_0, bias_0, ln_w_0, ln_b_0,
           basis_1, comp_1, root_1, bias_1, ln_w_1, ln_b_1):
    src, dst = edge_index[0], edge_index[1]
    seg = dst * R + edge_type
    cnt = jax.ops.segment_sum(jnp.ones((E,), jnp.float32), seg,
                              num_segments=N * R)
    w_e = (1.0 / jnp.clip(cnt, 1.0, None))[seg]
    gidx = src * R + edge_type

    h = jnp.where(valid_mask_flat[:, None], x_flat, 0.0)
    params = [(basis_0, comp_0, root_0, bias_0, ln_w_0, ln_b_0),
              (basis_1, comp_1, root_1, bias_1, ln_w_1, ln_b_1)]
    for (basis, comp, root, bias, ln_w, ln_b) in params:
        w = _mix(comp, basis)
        trans, hr = _trans(h, w, root)
        msg = trans.reshape(N * R, D)[gidx] * w_e[:, None]
        agg = jax.ops.segment_sum(msg, dst, num_segments=N)
        h = _post(agg, hr, h, bias, ln_w, ln_b)
    return jnp.where(valid_mask_flat[:, None], h, 0.0)


# speedup vs baseline: 3.4905x; 3.4905x over previous
import functools

import jax
import jax.numpy as jnp
from jax import lax
from jax.experimental import pallas as pl
from jax.experimental.pallas import tpu as pltpu
from jax.experimental.pallas import tpu_sc as plsc

N = 10000
D = 256
R = 8
NB = 8
E = 160000
EPS = 1e-5

BN = 1000
SEGR = 640
EPT = E // 16
CH = 80
NCH = EPT // CH
NPT = N // 16
MG = 2000


def _mix_kernel(comp_ref, basis_ref, w_ref):
    w_ref[...] = jax.lax.dot_general(
        comp_ref[...], basis_ref[...], (((1,), (0,)), ((), ())),
        preferred_element_type=jnp.float32)


def _mix(comp, basis):
    return pl.pallas_call(
        _mix_kernel,
        out_shape=jax.ShapeDtypeStruct((R, D * D), jnp.float32),
    )(comp, basis.reshape(NB, D * D)).reshape(R, D, D)


def _trans_kernel(h_ref, w_ref, root_ref, trans_ref, hr_ref):
    h = h_ref[...]
    t = jax.lax.dot_general(h, w_ref[...], (((1,), (1,)), ((), ())),
                            preferred_element_type=jnp.float32)
    trans_ref[...] = t.reshape(BN, R * D)
    hr_ref[...] = jnp.dot(h, root_ref[...], preferred_element_type=jnp.float32)


def _trans(h, w, root):
    grid = (N // BN,)
    return pl.pallas_call(
        _trans_kernel,
        grid=grid,
        in_specs=[
            pl.BlockSpec((BN, D), lambda i: (i, 0)),
            pl.BlockSpec((R, D, D), lambda i: (0, 0, 0)),
            pl.BlockSpec((D, D), lambda i: (0, 0)),
        ],
        out_specs=[
            pl.BlockSpec((BN, R * D), lambda i: (i, 0)),
            pl.BlockSpec((BN, D), lambda i: (i, 0)),
        ],
        out_shape=[
            jax.ShapeDtypeStruct((N, R * D), jnp.float32),
            jax.ShapeDtypeStruct((N, D), jnp.float32),
        ],
    )(h, w, root)


def _post_kernel(a0_ref, a1_ref, hr_ref, h_ref, bias_ref, lnw_ref, lnb_ref,
                 out_ref):
    agg = jnp.concatenate([a0_ref[...], a1_ref[...]], axis=1)
    m = agg + hr_ref[...] + bias_ref[...]
    m = 0.5 * m * (1.0 + jax.lax.erf(m * (2.0 ** -0.5)))
    x = h_ref[...] + m
    mu = jnp.mean(x, axis=-1, keepdims=True)
    var = jnp.mean((x - mu) ** 2, axis=-1, keepdims=True)
    out_ref[...] = (x - mu) / jnp.sqrt(var + EPS) * lnw_ref[...] + lnb_ref[...]


def _post(a0, a1, hr, h, bias, lnw, lnb):
    grid = (N // BN,)
    vec = lambda i: (0, 0)
    half = lambda i: (i, 0)
    return pl.pallas_call(
        _post_kernel,
        grid=grid,
        in_specs=[
            pl.BlockSpec((BN, 128), half),
            pl.BlockSpec((BN, 128), half),
            pl.BlockSpec((BN, D), half),
            pl.BlockSpec((BN, D), half),
            pl.BlockSpec((1, D), vec),
            pl.BlockSpec((1, D), vec),
            pl.BlockSpec((1, D), vec),
        ],
        out_specs=pl.BlockSpec((BN, D), lambda i: (i, 0)),
        out_shape=jax.ShapeDtypeStruct((N, D), jnp.float32),
    )(a0, a1, hr, h, bias.reshape(1, D), lnw.reshape(1, D), lnb.reshape(1, D))


def _sc_mesh():
    return plsc.VectorSubcoreMesh(core_axis_name="c", subcore_axis_name="s")


def _weights_body(seg_ref, w_ref, hist_v, seg_v, wv_v, ridx_v,
                  cnt_sh, sem):
    c = lax.axis_index("c")
    s = lax.axis_index("s")
    iota = lax.iota(jnp.int32, 16)
    z16 = jnp.zeros((16,), jnp.float32)
    ones = jnp.full((16,), 1.0, jnp.float32)

    @pl.when(c == 0)
    def _():
        def zb(i, _):
            for g in range(8):
                hist_v[i, pl.ds(g * 16, 16)] = z16
            return 0
        lax.fori_loop(0, SEGR, zb, 0)
        zoff = pl.multiple_of(s * 40, 8)
        pltpu.sync_copy(hist_v.at[pl.ds(0, 40)], cnt_sh.at[pl.ds(zoff, 40)])
        for k in range(5):
            for g in range(8):
                ridx_v[k, pl.ds(g * 16, 16)] = iota + (k * 128 + g * 16)
        base = pl.multiple_of(s * EPT, 8)
        pltpu.sync_copy(seg_ref.at[pl.ds(base, EPT)], seg_v)

        def ca(i, _):
            seg = seg_v[pl.ds(i * 16, 16)]
            plsc.addupdate_scatter(hist_v, [seg >> 7, seg & 127], ones)
            return 0
        lax.fori_loop(0, EPT // 16, ca, 0)
        plsc.subcore_barrier()
        for k in range(5):
            pltpu.sync_copy(hist_v.at[pl.ds(k * 128, 128)],
                            cnt_sh.at[ridx_v.at[k]], add=True)
        plsc.subcore_barrier()
        pltpu.sync_copy(cnt_sh, hist_v)

        def cw(i, _):
            seg = seg_v[pl.ds(i * 16, 16)]
            cntv = plsc.load_gather(hist_v, [seg >> 7, seg & 127])
            wv_v[pl.ds(i * 16, 16)] = 1.0 / jnp.maximum(cntv, 1.0)
            return 0
        lax.fori_loop(0, EPT // 16, cw, 0)
        pltpu.sync_copy(wv_v, w_ref.at[pl.ds(base, EPT)])


def _edge_weights(seg_e):
    k = pl.kernel(
        _weights_body,
        out_type=jax.ShapeDtypeStruct((E,), jnp.float32),
        mesh=_sc_mesh(),
        compiler_params=pltpu.CompilerParams(needs_layout_passes=False),
        scratch_types=[
            pltpu.VMEM((SEGR, 128), jnp.float32),
            pltpu.VMEM((EPT,), jnp.int32),
            pltpu.VMEM((EPT,), jnp.float32),
            pltpu.VMEM((5, 128), jnp.int32),
            pltpu.VMEM_SHARED((SEGR, 128), jnp.float32),
            pltpu.SemaphoreType.DMA,
        ],
    )
    return k(seg_e)


def _scatter_body(t2_ref, gi2_ref, dste_ref, we_ref, out0_ref, out1_ref,
                  gi8_v, dst8_v, w8_v, gi_v, di_v, rows_v, acc_sh, sem):
    c = lax.axis_index("c")
    s = lax.axis_index("s")
    z16 = jnp.zeros((16,), jnp.float32)
    base = pl.multiple_of(s * EPT, 8)

    def zb(i, _):
        for g in range(8):
            rows_v[i, pl.ds(g * 16, 16)] = z16
        return 0
    lax.fori_loop(0, CH, zb, 0)
    aoff = pl.multiple_of(s * 640, 8)
    for k in range(8):
        pltpu.sync_copy(rows_v, acc_sh.at[pl.ds(aoff + k * CH, CH)])
    plsc.subcore_barrier()

    for m in range(EPT // MG):
        moff = base + m * MG
        pltpu.sync_copy(gi2_ref.at[pl.ds(moff, MG)], gi8_v)
        pltpu.sync_copy(dste_ref.at[pl.ds(moff, MG)], dst8_v)
        pltpu.sync_copy(we_ref.at[pl.ds(moff, MG)], w8_v)

        def chunk(k, _):
            o = k * CH
            for g in range(5):
                sl = pl.ds(o + g * 16, 16)
                gi_v[pl.ds(g * 16, 16)] = gi8_v[sl] + c
                di_v[pl.ds(g * 16, 16)] = dst8_v[sl]
            pltpu.async_copy(t2_ref.at[gi_v], rows_v, sem).wait()

            def sc(g, _):
                w16 = w8_v[pl.ds(o + g * 16, 16)]
                for l in range(16):
                    w = w16[l]
                    for q in range(8):
                        sl2 = pl.ds(q * 16, 16)
                        rows_v[g * 16 + l, sl2] = rows_v[g * 16 + l, sl2] * w
                return 0
            lax.fori_loop(0, 5, sc, 0)
            pltpu.sync_copy(rows_v, acc_sh.at[di_v], add=True)
            return 0
        lax.fori_loop(0, MG // CH, chunk, 0)
    plsc.subcore_barrier()

    @pl.when(jnp.logical_and(c == 0, s < 15))
    def _():
        pltpu.sync_copy(acc_sh.at[pl.ds(aoff, 640)],
                        out0_ref.at[pl.ds(aoff, 640)])

    @pl.when(jnp.logical_and(c == 0, s == 15))
    def _():
        pltpu.sync_copy(acc_sh.at[pl.ds(9600, 400)],
                        out0_ref.at[pl.ds(9600, 400)])

    @pl.when(jnp.logical_and(c == 1, s < 15))
    def _():
        pltpu.sync_copy(acc_sh.at[pl.ds(aoff, 640)],
                        out1_ref.at[pl.ds(aoff, 640)])

    @pl.when(jnp.logical_and(c == 1, s == 15))
    def _():
        pltpu.sync_copy(acc_sh.at[pl.ds(9600, 400)],
                        out1_ref.at[pl.ds(9600, 400)])


def _gather_scatter(trans, gidx2, dst_e, w_e):
    t2 = trans.reshape(N * R * 2, 128)
    k = pl.kernel(
        _scatter_body,
        out_type=[
            jax.ShapeDtypeStruct((N, 128), jnp.float32),
            jax.ShapeDtypeStruct((N, 128), jnp.float32),
        ],
        mesh=_sc_mesh(),
        compiler_params=pltpu.CompilerParams(needs_layout_passes=False),
        scratch_types=[
            pltpu.VMEM((MG,), jnp.int32),
            pltpu.VMEM((MG,), jnp.int32),
            pltpu.VMEM((MG,), jnp.float32),
            pltpu.VMEM((CH,), jnp.int32),
            pltpu.VMEM((CH,), jnp.int32),
            pltpu.VMEM((CH, 128), jnp.float32),
            pltpu.VMEM_SHARED((10240, 128), jnp.float32),
            pltpu.SemaphoreType.DMA,
        ],
    )
    return k(t2, gidx2, dst_e, w_e)


def kernel(x_flat, edge_index, edge_type, valid_mask_flat,
           basis_0, comp_0, root_0, bias_0, ln_w_0, ln_b_0,
           basis_1, comp_1, root_1, bias_1, ln_w_1, ln_b_1):
    src_e, dst_e = edge_index[0], edge_index[1]
    seg_e = dst_e * R + edge_type
    gidx2 = src_e * (2 * R) + edge_type * 2
    w_e = _edge_weights(seg_e)

    h = jnp.where(valid_mask_flat[:, None], x_flat, 0.0)
    params = [(basis_0, comp_0, root_0, bias_0, ln_w_0, ln_b_0),
              (basis_1, comp_1, root_1, bias_1, ln_w_1, ln_b_1)]
    for (basis, comp, root, bias, ln_w, ln_b) in params:
        w = _mix(comp, basis)
        trans, hr = _trans(h, w, root)
        a0, a1 = _gather_scatter(trans, gidx2, dst_e, w_e)
        h = _post(a0, a1, hr, h, bias, ln_w, ln_b)
    return jnp.where(valid_mask_flat[:, None], h, 0.0)


# speedup vs baseline: 4.5392x; 1.3004x over previous
import functools

import jax
import jax.numpy as jnp
from jax import lax
from jax.experimental import pallas as pl
from jax.experimental.pallas import tpu as pltpu
from jax.experimental.pallas import tpu_sc as plsc

N = 10000
D = 256
R = 8
NB = 8
E = 160000
EPS = 1e-5

BN = 1000
SEGR = 640
EPT = E // 16
CH = 80
NCH = EPT // CH
NPT = N // 16
MG = 2000


def _mix_kernel(comp_ref, basis_ref, w_ref):
    w_ref[...] = jax.lax.dot_general(
        comp_ref[...], basis_ref[...], (((1,), (0,)), ((), ())),
        preferred_element_type=jnp.float32)


def _mix(comp, basis):
    return pl.pallas_call(
        _mix_kernel,
        out_shape=jax.ShapeDtypeStruct((R, D * D), jnp.float32),
    )(comp, basis.reshape(NB, D * D)).reshape(R, D, D)


def _trans_kernel(h_ref, w_ref, root_ref, trans_ref, hr_ref):
    h = h_ref[...]
    t = jax.lax.dot_general(h, w_ref[...], (((1,), (1,)), ((), ())),
                            preferred_element_type=jnp.float32)
    trans_ref[...] = t.reshape(BN, R * D)
    hr_ref[...] = jnp.dot(h, root_ref[...], preferred_element_type=jnp.float32)


def _trans(h, w, root):
    grid = (N // BN,)
    return pl.pallas_call(
        _trans_kernel,
        grid=grid,
        in_specs=[
            pl.BlockSpec((BN, D), lambda i: (i, 0)),
            pl.BlockSpec((R, D, D), lambda i: (0, 0, 0)),
            pl.BlockSpec((D, D), lambda i: (0, 0)),
        ],
        out_specs=[
            pl.BlockSpec((BN, R * D), lambda i: (i, 0)),
            pl.BlockSpec((BN, D), lambda i: (i, 0)),
        ],
        out_shape=[
            jax.ShapeDtypeStruct((N, R * D), jnp.float32),
            jax.ShapeDtypeStruct((N, D), jnp.float32),
        ],
    )(h, w, root)


def _post_kernel(a0_ref, a1_ref, hr_ref, h_ref, bias_ref, lnw_ref, lnb_ref,
                 out_ref):
    agg = jnp.concatenate([a0_ref[...], a1_ref[...]], axis=1)
    m = agg + hr_ref[...] + bias_ref[...]
    m = 0.5 * m * (1.0 + jax.lax.erf(m * (2.0 ** -0.5)))
    x = h_ref[...] + m
    mu = jnp.mean(x, axis=-1, keepdims=True)
    var = jnp.mean((x - mu) ** 2, axis=-1, keepdims=True)
    out_ref[...] = (x - mu) / jnp.sqrt(var + EPS) * lnw_ref[...] + lnb_ref[...]


def _post(a0, a1, hr, h, bias, lnw, lnb):
    grid = (N // BN,)
    vec = lambda i: (0, 0)
    half = lambda i: (i, 0)
    return pl.pallas_call(
        _post_kernel,
        grid=grid,
        in_specs=[
            pl.BlockSpec((BN, 128), half),
            pl.BlockSpec((BN, 128), half),
            pl.BlockSpec((BN, D), half),
            pl.BlockSpec((BN, D), half),
            pl.BlockSpec((1, D), vec),
            pl.BlockSpec((1, D), vec),
            pl.BlockSpec((1, D), vec),
        ],
        out_specs=pl.BlockSpec((BN, D), lambda i: (i, 0)),
        out_shape=jax.ShapeDtypeStruct((N, D), jnp.float32),
    )(a0, a1, hr, h, bias.reshape(1, D), lnw.reshape(1, D), lnb.reshape(1, D))


def _sc_mesh():
    return plsc.VectorSubcoreMesh(core_axis_name="c", subcore_axis_name="s")


def _weights_body(seg_ref, w_ref, hist_v, seg_v, wv_v, ridx_v,
                  cnt_sh, sem):
    c = lax.axis_index("c")
    s = lax.axis_index("s")
    iota = lax.iota(jnp.int32, 16)
    z16 = jnp.zeros((16,), jnp.float32)
    ones = jnp.full((16,), 1.0, jnp.float32)

    @pl.when(c == 0)
    def _():
        def zb(i, _):
            for g in range(8):
                hist_v[i, pl.ds(g * 16, 16)] = z16
            return 0
        lax.fori_loop(0, SEGR, zb, 0)
        zoff = pl.multiple_of(s * 40, 8)
        pltpu.sync_copy(hist_v.at[pl.ds(0, 40)], cnt_sh.at[pl.ds(zoff, 40)])
        for k in range(5):
            for g in range(8):
                ridx_v[k, pl.ds(g * 16, 16)] = iota + (k * 128 + g * 16)
        base = pl.multiple_of(s * EPT, 8)
        pltpu.sync_copy(seg_ref.at[pl.ds(base, EPT)], seg_v)

        def ca(i, _):
            seg = seg_v[pl.ds(i * 16, 16)]
            plsc.addupdate_scatter(hist_v, [seg >> 7, seg & 127], ones)
            return 0
        lax.fori_loop(0, EPT // 16, ca, 0)
        plsc.subcore_barrier()
        for k in range(5):
            pltpu.sync_copy(hist_v.at[pl.ds(k * 128, 128)],
                            cnt_sh.at[ridx_v.at[k]], add=True)
        plsc.subcore_barrier()
        pltpu.sync_copy(cnt_sh, hist_v)

        def cw(i, _):
            seg = seg_v[pl.ds(i * 16, 16)]
            cntv = plsc.load_gather(hist_v, [seg >> 7, seg & 127])
            wv_v[pl.ds(i * 16, 16)] = 1.0 / jnp.maximum(cntv, 1.0)
            return 0
        lax.fori_loop(0, EPT // 16, cw, 0)
        pltpu.sync_copy(wv_v, w_ref.at[pl.ds(base, EPT)])


def _edge_weights(seg_e):
    k = pl.kernel(
        _weights_body,
        out_type=jax.ShapeDtypeStruct((E,), jnp.float32),
        mesh=_sc_mesh(),
        compiler_params=pltpu.CompilerParams(needs_layout_passes=False),
        scratch_types=[
            pltpu.VMEM((SEGR, 128), jnp.float32),
            pltpu.VMEM((EPT,), jnp.int32),
            pltpu.VMEM((EPT,), jnp.float32),
            pltpu.VMEM((5, 128), jnp.int32),
            pltpu.VMEM_SHARED((SEGR, 128), jnp.float32),
            pltpu.SemaphoreType.DMA,
        ],
    )
    return k(seg_e)


def _scatter_body(t2_ref, meta_ref, we_ref, out0_ref, out1_ref,
                  mv_v, wv_v, giA, diA, giB, diB, rowsA, rowsB,
                  acc_sh, gsA, gsB, ssA, ssB):
    c = lax.axis_index("c")
    s = lax.axis_index("s")
    z16 = jnp.zeros((16,), jnp.float32)
    base = pl.multiple_of(s * EPT, 8)

    def idx_chunk(o, gi_ref, di_ref):
        for g in range(5):
            m16 = mv_v[pl.ds(o + g * 16, 16)]
            gi_ref[pl.ds(g * 16, 16)] = lax.shift_right_logical(m16, 14) + c
            di_ref[pl.ds(g * 16, 16)] = m16 & 16383

    def scale(rows_ref, o):
        def sc(g, _):
            w16 = wv_v[pl.ds(o + g * 16, 16)]
            for l in range(16):
                w = w16[l]
                for q in range(8):
                    sl2 = pl.ds(q * 16, 16)
                    rows_ref[g * 16 + l, sl2] = rows_ref[g * 16 + l, sl2] * w
            return 0
        lax.fori_loop(0, 5, sc, 0)

    def gather(gi_ref, rows_ref, sem):
        pltpu.async_copy(t2_ref.at[gi_ref], rows_ref, sem)

    def wait_g(gi_ref, rows_ref, sem):
        pltpu.make_async_copy(t2_ref.at[gi_ref], rows_ref, sem).wait()

    def scatter(rows_ref, di_ref, sem):
        pltpu.async_copy(rows_ref, acc_sh.at[di_ref], sem, add=True)

    def wait_s(rows_ref, di_ref, sem):
        pltpu.make_async_copy(rows_ref, acc_sh.at[di_ref], sem).wait()

    def zb(i, _):
        for g in range(8):
            rowsA[i, pl.ds(g * 16, 16)] = z16
        return 0
    lax.fori_loop(0, CH, zb, 0)
    aoff = pl.multiple_of(s * 640, 8)
    for k in range(8):
        pltpu.sync_copy(rowsA, acc_sh.at[pl.ds(aoff + k * CH, CH)])
    plsc.subcore_barrier()

    for m in range(EPT // MG):
        moff = base + m * MG
        pltpu.sync_copy(meta_ref.at[pl.ds(moff, MG)], mv_v)
        pltpu.sync_copy(we_ref.at[pl.ds(moff, MG)], wv_v)

        idx_chunk(0, giA, diA)
        gather(giA, rowsA, gsA)

        def pair(i, _):
            oa = i * (2 * CH)
            ob = oa + CH
            oa2 = oa + 2 * CH
            @pl.when(i > 0)
            def _():
                wait_s(rowsB, diB, ssB)
            idx_chunk(ob, giB, diB)
            gather(giB, rowsB, gsB)
            wait_g(giA, rowsA, gsA)
            scale(rowsA, oa)
            scatter(rowsA, diA, ssA)
            wait_g(giB, rowsB, gsB)
            scale(rowsB, ob)
            wait_s(rowsA, diA, ssA)
            idx_chunk(oa2, giA, diA)
            gather(giA, rowsA, gsA)
            scatter(rowsB, diB, ssB)
            return 0
        lax.fori_loop(0, (MG // CH - 1) // 2, pair, 0)

        olast = (MG // CH - 1) * CH
        wait_g(giA, rowsA, gsA)
        scale(rowsA, olast)
        pltpu.sync_copy(rowsA, acc_sh.at[diA], add=True)
        wait_s(rowsB, diB, ssB)
    plsc.subcore_barrier()

    @pl.when(jnp.logical_and(c == 0, s < 15))
    def _():
        pltpu.sync_copy(acc_sh.at[pl.ds(aoff, 640)],
                        out0_ref.at[pl.ds(aoff, 640)])

    @pl.when(jnp.logical_and(c == 0, s == 15))
    def _():
        pltpu.sync_copy(acc_sh.at[pl.ds(9600, 400)],
                        out0_ref.at[pl.ds(9600, 400)])

    @pl.when(jnp.logical_and(c == 1, s < 15))
    def _():
        pltpu.sync_copy(acc_sh.at[pl.ds(aoff, 640)],
                        out1_ref.at[pl.ds(aoff, 640)])

    @pl.when(jnp.logical_and(c == 1, s == 15))
    def _():
        pltpu.sync_copy(acc_sh.at[pl.ds(9600, 400)],
                        out1_ref.at[pl.ds(9600, 400)])


def _gather_scatter(trans, meta_e, w_e):
    t2 = trans.reshape(N * R * 2, 128)
    k = pl.kernel(
        _scatter_body,
        out_type=[
            jax.ShapeDtypeStruct((N, 128), jnp.float32),
            jax.ShapeDtypeStruct((N, 128), jnp.float32),
        ],
        mesh=_sc_mesh(),
        compiler_params=pltpu.CompilerParams(needs_layout_passes=False),
        scratch_types=[
            pltpu.VMEM((MG,), jnp.int32),
            pltpu.VMEM((MG,), jnp.float32),
            pltpu.VMEM((CH,), jnp.int32),
            pltpu.VMEM((CH,), jnp.int32),
            pltpu.VMEM((CH,), jnp.int32),
            pltpu.VMEM((CH,), jnp.int32),
            pltpu.VMEM((CH, 128), jnp.float32),
            pltpu.VMEM((CH, 128), jnp.float32),
            pltpu.VMEM_SHARED((10240, 128), jnp.float32),
            pltpu.SemaphoreType.DMA,
            pltpu.SemaphoreType.DMA,
            pltpu.SemaphoreType.DMA,
            pltpu.SemaphoreType.DMA,
        ],
    )
    return k(t2, meta_e, w_e)


def kernel(x_flat, edge_index, edge_type, valid_mask_flat,
           basis_0, comp_0, root_0, bias_0, ln_w_0, ln_b_0,
           basis_1, comp_1, root_1, bias_1, ln_w_1, ln_b_1):
    src_e, dst_e = edge_index[0], edge_index[1]
    seg_e = dst_e * R + edge_type
    gidx2 = src_e * (2 * R) + edge_type * 2
    meta_e = (gidx2 << 14) | dst_e
    w_e = _edge_weights(seg_e)

    h = jnp.where(valid_mask_flat[:, None], x_flat, 0.0)
    params = [(basis_0, comp_0, root_0, bias_0, ln_w_0, ln_b_0),
              (basis_1, comp_1, root_1, bias_1, ln_w_1, ln_b_1)]
    for (basis, comp, root, bias, ln_w, ln_b) in params:
        w = _mix(comp, basis)
        trans, hr = _trans(h, w, root)
        a0, a1 = _gather_scatter(trans, meta_e, w_e)
        h = _post(a0, a1, hr, h, bias, ln_w, ln_b)
    return jnp.where(valid_mask_flat[:, None], h, 0.0)


# speedup vs baseline: 4.6285x; 1.0197x over previous
import functools

import jax
import jax.numpy as jnp
from jax import lax
from jax.experimental import pallas as pl
from jax.experimental.pallas import tpu as pltpu
from jax.experimental.pallas import tpu_sc as plsc

N = 10000
D = 256
R = 8
NB = 8
E = 160000
EPS = 1e-5

BN = 1000
SEGR = 640
EPT = E // 16
CH = 80
NCH = EPT // CH
NPT = N // 16
MG = 2000


def _mix_kernel(comp_ref, basis_ref, w_ref):
    w_ref[...] = jax.lax.dot_general(
        comp_ref[...], basis_ref[...], (((1,), (0,)), ((), ())),
        preferred_element_type=jnp.float32)


def _mix(comp, basis):
    return pl.pallas_call(
        _mix_kernel,
        out_shape=jax.ShapeDtypeStruct((R, D * D), jnp.float32),
    )(comp, basis.reshape(NB, D * D)).reshape(R, D, D)


def _trans_kernel(h_ref, w_ref, root_ref, trans_ref, hr_ref):
    h = h_ref[...]
    t = jax.lax.dot_general(h, w_ref[...], (((1,), (1,)), ((), ())),
                            preferred_element_type=jnp.float32)
    trans_ref[...] = t.reshape(BN, R * D)
    hr_ref[...] = jnp.dot(h, root_ref[...], preferred_element_type=jnp.float32)


def _trans(h, w, root):
    grid = (N // BN,)
    return pl.pallas_call(
        _trans_kernel,
        grid=grid,
        in_specs=[
            pl.BlockSpec((BN, D), lambda i: (i, 0)),
            pl.BlockSpec((R, D, D), lambda i: (0, 0, 0)),
            pl.BlockSpec((D, D), lambda i: (0, 0)),
        ],
        out_specs=[
            pl.BlockSpec((BN, R * D), lambda i: (i, 0)),
            pl.BlockSpec((BN, D), lambda i: (i, 0)),
        ],
        out_shape=[
            jax.ShapeDtypeStruct((N, R * D), jnp.float32),
            jax.ShapeDtypeStruct((N, D), jnp.float32),
        ],
    )(h, w, root)


def _post_kernel(a0_ref, a1_ref, hr_ref, h_ref, bias_ref, lnw_ref, lnb_ref,
                 out_ref):
    agg = jnp.concatenate([a0_ref[...], a1_ref[...]], axis=1)
    m = agg + hr_ref[...] + bias_ref[...]
    m = 0.5 * m * (1.0 + jax.lax.erf(m * (2.0 ** -0.5)))
    x = h_ref[...] + m
    mu = jnp.mean(x, axis=-1, keepdims=True)
    var = jnp.mean((x - mu) ** 2, axis=-1, keepdims=True)
    out_ref[...] = (x - mu) / jnp.sqrt(var + EPS) * lnw_ref[...] + lnb_ref[...]


def _post(a0, a1, hr, h, bias, lnw, lnb):
    grid = (N // BN,)
    vec = lambda i: (0, 0)
    half = lambda i: (i, 0)
    return pl.pallas_call(
        _post_kernel,
        grid=grid,
        in_specs=[
            pl.BlockSpec((BN, 128), half),
            pl.BlockSpec((BN, 128), half),
            pl.BlockSpec((BN, D), half),
            pl.BlockSpec((BN, D), half),
            pl.BlockSpec((1, D), vec),
            pl.BlockSpec((1, D), vec),
            pl.BlockSpec((1, D), vec),
        ],
        out_specs=pl.BlockSpec((BN, D), lambda i: (i, 0)),
        out_shape=jax.ShapeDtypeStruct((N, D), jnp.float32),
    )(a0, a1, hr, h, bias.reshape(1, D), lnw.reshape(1, D), lnb.reshape(1, D))


def _post_trans_kernel(a0_ref, a1_ref, hr_ref, h_ref, bias_ref, lnw_ref,
                       lnb_ref, w_ref, root_ref, hn_ref, trans_ref, hr2_ref):
    agg = jnp.concatenate([a0_ref[...], a1_ref[...]], axis=1)
    m = agg + hr_ref[...] + bias_ref[...]
    m = 0.5 * m * (1.0 + jax.lax.erf(m * (2.0 ** -0.5)))
    x = h_ref[...] + m
    mu = jnp.mean(x, axis=-1, keepdims=True)
    var = jnp.mean((x - mu) ** 2, axis=-1, keepdims=True)
    hn = (x - mu) / jnp.sqrt(var + EPS) * lnw_ref[...] + lnb_ref[...]
    hn_ref[...] = hn
    t = jax.lax.dot_general(hn, w_ref[...], (((1,), (1,)), ((), ())),
                            preferred_element_type=jnp.float32)
    trans_ref[...] = t.reshape(BN, R * D)
    hr2_ref[...] = jnp.dot(hn, root_ref[...],
                           preferred_element_type=jnp.float32)


def _post_trans(a0, a1, hr, h, bias, lnw, lnb, w2, root2):
    grid = (N // BN,)
    vec = lambda i: (0, 0)
    half = lambda i: (i, 0)
    return pl.pallas_call(
        _post_trans_kernel,
        grid=grid,
        in_specs=[
            pl.BlockSpec((BN, 128), half),
            pl.BlockSpec((BN, 128), half),
            pl.BlockSpec((BN, D), half),
            pl.BlockSpec((BN, D), half),
            pl.BlockSpec((1, D), vec),
            pl.BlockSpec((1, D), vec),
            pl.BlockSpec((1, D), vec),
            pl.BlockSpec((R, D, D), lambda i: (0, 0, 0)),
            pl.BlockSpec((D, D), vec),
        ],
        out_specs=[
            pl.BlockSpec((BN, D), half),
            pl.BlockSpec((BN, R * D), half),
            pl.BlockSpec((BN, D), half),
        ],
        out_shape=[
            jax.ShapeDtypeStruct((N, D), jnp.float32),
            jax.ShapeDtypeStruct((N, R * D), jnp.float32),
            jax.ShapeDtypeStruct((N, D), jnp.float32),
        ],
    )(a0, a1, hr, h, bias.reshape(1, D), lnw.reshape(1, D),
      lnb.reshape(1, D), w2, root2)


def _sc_mesh():
    return plsc.VectorSubcoreMesh(core_axis_name="c", subcore_axis_name="s")


def _weights_body(seg_ref, w_ref, hist_v, seg_v, wv_v, ridx_v,
                  cnt_sh, sem):
    c = lax.axis_index("c")
    s = lax.axis_index("s")
    iota = lax.iota(jnp.int32, 16)
    z16 = jnp.zeros((16,), jnp.float32)
    ones = jnp.full((16,), 1.0, jnp.float32)

    @pl.when(c == 0)
    def _():
        def zb(i, _):
            for g in range(8):
                hist_v[i, pl.ds(g * 16, 16)] = z16
            return 0
        lax.fori_loop(0, SEGR, zb, 0)
        zoff = pl.multiple_of(s * 40, 8)
        pltpu.sync_copy(hist_v.at[pl.ds(0, 40)], cnt_sh.at[pl.ds(zoff, 40)])
        for k in range(5):
            for g in range(8):
                ridx_v[k, pl.ds(g * 16, 16)] = iota + (k * 128 + g * 16)
        base = pl.multiple_of(s * EPT, 8)
        pltpu.sync_copy(seg_ref.at[pl.ds(base, EPT)], seg_v)

        def ca(i, _):
            seg = seg_v[pl.ds(i * 16, 16)]
            plsc.addupdate_scatter(hist_v, [seg >> 7, seg & 127], ones)
            return 0
        lax.fori_loop(0, EPT // 16, ca, 0)
        plsc.subcore_barrier()
        for k in range(5):
            pltpu.sync_copy(hist_v.at[pl.ds(k * 128, 128)],
                            cnt_sh.at[ridx_v.at[k]], add=True)
        plsc.subcore_barrier()
        pltpu.sync_copy(cnt_sh, hist_v)

        def cw(i, _):
            seg = seg_v[pl.ds(i * 16, 16)]
            cntv = plsc.load_gather(hist_v, [seg >> 7, seg & 127])
            wv_v[pl.ds(i * 16, 16)] = 1.0 / jnp.maximum(cntv, 1.0)
            return 0
        lax.fori_loop(0, EPT // 16, cw, 0)
        pltpu.sync_copy(wv_v, w_ref.at[pl.ds(base, EPT)])


def _edge_weights(seg_e):
    k = pl.kernel(
        _weights_body,
        out_type=jax.ShapeDtypeStruct((E,), jnp.float32),
        mesh=_sc_mesh(),
        compiler_params=pltpu.CompilerParams(needs_layout_passes=False),
        scratch_types=[
            pltpu.VMEM((SEGR, 128), jnp.float32),
            pltpu.VMEM((EPT,), jnp.int32),
            pltpu.VMEM((EPT,), jnp.float32),
            pltpu.VMEM((5, 128), jnp.int32),
            pltpu.VMEM_SHARED((SEGR, 128), jnp.float32),
            pltpu.SemaphoreType.DMA,
        ],
    )
    return k(seg_e)


def _scatter_body(t2_ref, meta_ref, we_ref, out0_ref, out1_ref,
                  mv_v, wv_v, giA, diA, giB, diB, rowsA, rowsB,
                  acc_sh, gsA, gsB, ssA, ssB):
    c = lax.axis_index("c")
    s = lax.axis_index("s")
    z16 = jnp.zeros((16,), jnp.float32)
    base = pl.multiple_of(s * EPT, 8)

    def idx_chunk(o, gi_ref, di_ref):
        for g in range(5):
            m16 = mv_v[pl.ds(o + g * 16, 16)]
            gi_ref[pl.ds(g * 16, 16)] = lax.shift_right_logical(m16, 14) + c
            di_ref[pl.ds(g * 16, 16)] = m16 & 16383

    def scale(rows_ref, o):
        def sc(g, _):
            w16 = wv_v[pl.ds(o + g * 16, 16)]
            for l in range(16):
                w = w16[l]
                for q in range(8):
                    sl2 = pl.ds(q * 16, 16)
                    rows_ref[g * 16 + l, sl2] = rows_ref[g * 16 + l, sl2] * w
            return 0
        lax.fori_loop(0, 5, sc, 0)

    def gather(gi_ref, rows_ref, sem):
        pltpu.async_copy(t2_ref.at[gi_ref], rows_ref, sem)

    def wait_g(gi_ref, rows_ref, sem):
        pltpu.make_async_copy(t2_ref.at[gi_ref], rows_ref, sem).wait()

    def scatter(rows_ref, di_ref, sem):
        pltpu.async_copy(rows_ref, acc_sh.at[di_ref], sem, add=True)

    def wait_s(rows_ref, di_ref, sem):
        pltpu.make_async_copy(rows_ref, acc_sh.at[di_ref], sem).wait()

    def zb(i, _):
        for g in range(8):
            rowsA[i, pl.ds(g * 16, 16)] = z16
        return 0
    lax.fori_loop(0, CH, zb, 0)
    aoff = pl.multiple_of(s * 640, 8)
    for k in range(8):
        pltpu.sync_copy(rowsA, acc_sh.at[pl.ds(aoff + k * CH, CH)])
    plsc.subcore_barrier()

    for m in range(EPT // MG):
        moff = base + m * MG
        pltpu.sync_copy(meta_ref.at[pl.ds(moff, MG)], mv_v)
        pltpu.sync_copy(we_ref.at[pl.ds(moff, MG)], wv_v)

        idx_chunk(0, giA, diA)
        gather(giA, rowsA, gsA)

        def pair(i, _):
            oa = i * (2 * CH)
            ob = oa + CH
            oa2 = oa + 2 * CH
            @pl.when(i > 0)
            def _():
                wait_s(rowsB, diB, ssB)
            idx_chunk(ob, giB, diB)
            gather(giB, rowsB, gsB)
            wait_g(giA, rowsA, gsA)
            scale(rowsA, oa)
            scatter(rowsA, diA, ssA)
            wait_g(giB, rowsB, gsB)
            scale(rowsB, ob)
            wait_s(rowsA, diA, ssA)
            idx_chunk(oa2, giA, diA)
            gather(giA, rowsA, gsA)
            scatter(rowsB, diB, ssB)
            return 0
        lax.fori_loop(0, (MG // CH - 1) // 2, pair, 0)

        olast = (MG // CH - 1) * CH
        wait_g(giA, rowsA, gsA)
        scale(rowsA, olast)
        pltpu.sync_copy(rowsA, acc_sh.at[diA], add=True)
        wait_s(rowsB, diB, ssB)
    plsc.subcore_barrier()

    @pl.when(jnp.logical_and(c == 0, s < 15))
    def _():
        pltpu.sync_copy(acc_sh.at[pl.ds(aoff, 640)],
                        out0_ref.at[pl.ds(aoff, 640)])

    @pl.when(jnp.logical_and(c == 0, s == 15))
    def _():
        pltpu.sync_copy(acc_sh.at[pl.ds(9600, 400)],
                        out0_ref.at[pl.ds(9600, 400)])

    @pl.when(jnp.logical_and(c == 1, s < 15))
    def _():
        pltpu.sync_copy(acc_sh.at[pl.ds(aoff, 640)],
                        out1_ref.at[pl.ds(aoff, 640)])

    @pl.when(jnp.logical_and(c == 1, s == 15))
    def _():
        pltpu.sync_copy(acc_sh.at[pl.ds(9600, 400)],
                        out1_ref.at[pl.ds(9600, 400)])


def _gather_scatter(trans, meta_e, w_e):
    t2 = trans.reshape(N * R * 2, 128)
    k = pl.kernel(
        _scatter_body,
        out_type=[
            jax.ShapeDtypeStruct((N, 128), jnp.float32),
            jax.ShapeDtypeStruct((N, 128), jnp.float32),
        ],
        mesh=_sc_mesh(),
        compiler_params=pltpu.CompilerParams(needs_layout_passes=False),
        scratch_types=[
            pltpu.VMEM((MG,), jnp.int32),
            pltpu.VMEM((MG,), jnp.float32),
            pltpu.VMEM((CH,), jnp.int32),
            pltpu.VMEM((CH,), jnp.int32),
            pltpu.VMEM((CH,), jnp.int32),
            pltpu.VMEM((CH,), jnp.int32),
            pltpu.VMEM((CH, 128), jnp.float32),
            pltpu.VMEM((CH, 128), jnp.float32),
            pltpu.VMEM_SHARED((10240, 128), jnp.float32),
            pltpu.SemaphoreType.DMA,
            pltpu.SemaphoreType.DMA,
            pltpu.SemaphoreType.DMA,
            pltpu.SemaphoreType.DMA,
        ],
    )
    return k(t2, meta_e, w_e)


def kernel(x_flat, edge_index, edge_type, valid_mask_flat,
           basis_0, comp_0, root_0, bias_0, ln_w_0, ln_b_0,
           basis_1, comp_1, root_1, bias_1, ln_w_1, ln_b_1):
    src_e, dst_e = edge_index[0], edge_index[1]
    seg_e = dst_e * R + edge_type
    gidx2 = src_e * (2 * R) + edge_type * 2
    meta_e = (gidx2 << 14) | dst_e
    w_e = _edge_weights(seg_e)

    h = jnp.where(valid_mask_flat[:, None], x_flat, 0.0)
    w1 = _mix(comp_0, basis_0)
    w2 = _mix(comp_1, basis_1)
    trans, hr = _trans(h, w1, root_0)
    a0, a1 = _gather_scatter(trans, meta_e, w_e)
    h1, trans2, hr2 = _post_trans(a0, a1, hr, h, bias_0, ln_w_0, ln_b_0,
                                  w2, root_1)
    b0, b1 = _gather_scatter(trans2, meta_e, w_e)
    h2 = _post(b0, b1, hr2, h1, bias_1, ln_w_1, ln_b_1)
    return jnp.where(valid_mask_flat[:, None], h2, 0.0)


# speedup vs baseline: 4.6388x; 1.0022x over previous
import functools

import jax
import jax.numpy as jnp
from jax import lax
from jax.experimental import pallas as pl
from jax.experimental.pallas import tpu as pltpu
from jax.experimental.pallas import tpu_sc as plsc

N = 10000
D = 256
R = 8
NB = 8
E = 160000
EPS = 1e-5

BN = 1000
SEGR = 640
EPT = E // 16
CH = 80
NCH = EPT // CH
NPT = N // 16
MG = 2000


def _mix_kernel(comp_ref, basis_ref, w_ref):
    w_ref[...] = jax.lax.dot_general(
        comp_ref[...], basis_ref[...], (((1,), (0,)), ((), ())),
        preferred_element_type=jnp.float32)


def _mix(comp, basis):
    return pl.pallas_call(
        _mix_kernel,
        out_shape=jax.ShapeDtypeStruct((R, D * D), jnp.float32),
    )(comp, basis.reshape(NB, D * D)).reshape(R, D, D)


def _trans_kernel(h_ref, w_ref, root_ref, trans_ref, hr_ref):
    h = h_ref[...].astype(jnp.bfloat16)
    t = jax.lax.dot_general(h, w_ref[...], (((1,), (1,)), ((), ())),
                            preferred_element_type=jnp.float32)
    trans_ref[...] = t.reshape(BN, R * D)
    hr_ref[...] = jnp.dot(h, root_ref[...], preferred_element_type=jnp.float32)


def _trans(h, w, root):
    grid = (N // BN,)
    return pl.pallas_call(
        _trans_kernel,
        grid=grid,
        in_specs=[
            pl.BlockSpec((BN, D), lambda i: (i, 0)),
            pl.BlockSpec((R, D, D), lambda i: (0, 0, 0)),
            pl.BlockSpec((D, D), lambda i: (0, 0)),
        ],
        out_specs=[
            pl.BlockSpec((BN, R * D), lambda i: (i, 0)),
            pl.BlockSpec((BN, D), lambda i: (i, 0)),
        ],
        out_shape=[
            jax.ShapeDtypeStruct((N, R * D), jnp.float32),
            jax.ShapeDtypeStruct((N, D), jnp.float32),
        ],
    )(h, w, root)


def _post_kernel(a0_ref, a1_ref, hr_ref, h_ref, bias_ref, lnw_ref, lnb_ref,
                 out_ref):
    agg = jnp.concatenate([a0_ref[...], a1_ref[...]], axis=1)
    m = agg + hr_ref[...] + bias_ref[...]
    m = 0.5 * m * (1.0 + jax.lax.erf(m * (2.0 ** -0.5)))
    x = h_ref[...] + m
    mu = jnp.mean(x, axis=-1, keepdims=True)
    var = jnp.mean((x - mu) ** 2, axis=-1, keepdims=True)
    out_ref[...] = (x - mu) / jnp.sqrt(var + EPS) * lnw_ref[...] + lnb_ref[...]


def _post(a0, a1, hr, h, bias, lnw, lnb):
    grid = (N // BN,)
    vec = lambda i: (0, 0)
    half = lambda i: (i, 0)
    return pl.pallas_call(
        _post_kernel,
        grid=grid,
        in_specs=[
            pl.BlockSpec((BN, 128), half),
            pl.BlockSpec((BN, 128), half),
            pl.BlockSpec((BN, D), half),
            pl.BlockSpec((BN, D), half),
            pl.BlockSpec((1, D), vec),
            pl.BlockSpec((1, D), vec),
            pl.BlockSpec((1, D), vec),
        ],
        out_specs=pl.BlockSpec((BN, D), lambda i: (i, 0)),
        out_shape=jax.ShapeDtypeStruct((N, D), jnp.float32),
    )(a0, a1, hr, h, bias.reshape(1, D), lnw.reshape(1, D), lnb.reshape(1, D))


def _post_trans_kernel(a0_ref, a1_ref, hr_ref, h_ref, bias_ref, lnw_ref,
                       lnb_ref, w_ref, root_ref, hn_ref, trans_ref, hr2_ref):
    agg = jnp.concatenate([a0_ref[...], a1_ref[...]], axis=1)
    m = agg + hr_ref[...] + bias_ref[...]
    m = 0.5 * m * (1.0 + jax.lax.erf(m * (2.0 ** -0.5)))
    x = h_ref[...] + m
    mu = jnp.mean(x, axis=-1, keepdims=True)
    var = jnp.mean((x - mu) ** 2, axis=-1, keepdims=True)
    hn = (x - mu) / jnp.sqrt(var + EPS) * lnw_ref[...] + lnb_ref[...]
    hn_ref[...] = hn
    hb = hn.astype(jnp.bfloat16)
    t = jax.lax.dot_general(hb, w_ref[...], (((1,), (1,)), ((), ())),
                            preferred_element_type=jnp.float32)
    trans_ref[...] = t.reshape(BN, R * D)
    hr2_ref[...] = jnp.dot(hb, root_ref[...],
                           preferred_element_type=jnp.float32)


def _post_trans(a0, a1, hr, h, bias, lnw, lnb, w2, root2):
    grid = (N // BN,)
    vec = lambda i: (0, 0)
    half = lambda i: (i, 0)
    return pl.pallas_call(
        _post_trans_kernel,
        grid=grid,
        in_specs=[
            pl.BlockSpec((BN, 128), half),
            pl.BlockSpec((BN, 128), half),
            pl.BlockSpec((BN, D), half),
            pl.BlockSpec((BN, D), half),
            pl.BlockSpec((1, D), vec),
            pl.BlockSpec((1, D), vec),
            pl.BlockSpec((1, D), vec),
            pl.BlockSpec((R, D, D), lambda i: (0, 0, 0)),
            pl.BlockSpec((D, D), vec),
        ],
        out_specs=[
            pl.BlockSpec((BN, D), half),
            pl.BlockSpec((BN, R * D), half),
            pl.BlockSpec((BN, D), half),
        ],
        out_shape=[
            jax.ShapeDtypeStruct((N, D), jnp.float32),
            jax.ShapeDtypeStruct((N, R * D), jnp.float32),
            jax.ShapeDtypeStruct((N, D), jnp.float32),
        ],
    )(a0, a1, hr, h, bias.reshape(1, D), lnw.reshape(1, D),
      lnb.reshape(1, D), w2, root2)


def _sc_mesh():
    return plsc.VectorSubcoreMesh(core_axis_name="c", subcore_axis_name="s")


def _weights_body(seg_ref, w_ref, hist_v, seg_v, wv_v, ridx_v,
                  cnt_sh, sem):
    c = lax.axis_index("c")
    s = lax.axis_index("s")
    iota = lax.iota(jnp.int32, 16)
    z16 = jnp.zeros((16,), jnp.float32)
    ones = jnp.full((16,), 1.0, jnp.float32)

    @pl.when(c == 0)
    def _():
        def zb(i, _):
            for g in range(8):
                hist_v[i, pl.ds(g * 16, 16)] = z16
            return 0
        lax.fori_loop(0, SEGR, zb, 0)
        zoff = pl.multiple_of(s * 40, 8)
        pltpu.sync_copy(hist_v.at[pl.ds(0, 40)], cnt_sh.at[pl.ds(zoff, 40)])
        for k in range(5):
            for g in range(8):
                ridx_v[k, pl.ds(g * 16, 16)] = iota + (k * 128 + g * 16)
        base = pl.multiple_of(s * EPT, 8)
        pltpu.sync_copy(seg_ref.at[pl.ds(base, EPT)], seg_v)

        def ca(i, _):
            seg = seg_v[pl.ds(i * 16, 16)]
            plsc.addupdate_scatter(hist_v, [seg >> 7, seg & 127], ones)
            return 0
        lax.fori_loop(0, EPT // 16, ca, 0)
        plsc.subcore_barrier()
        for k in range(5):
            pltpu.sync_copy(hist_v.at[pl.ds(k * 128, 128)],
                            cnt_sh.at[ridx_v.at[k]], add=True)
        plsc.subcore_barrier()
        pltpu.sync_copy(cnt_sh, hist_v)

        def cw(i, _):
            seg = seg_v[pl.ds(i * 16, 16)]
            cntv = plsc.load_gather(hist_v, [seg >> 7, seg & 127])
            wv_v[pl.ds(i * 16, 16)] = 1.0 / jnp.maximum(cntv, 1.0)
            return 0
        lax.fori_loop(0, EPT // 16, cw, 0)
        pltpu.sync_copy(wv_v, w_ref.at[pl.ds(base, EPT)])


def _edge_weights(seg_e):
    k = pl.kernel(
        _weights_body,
        out_type=jax.ShapeDtypeStruct((E,), jnp.float32),
        mesh=_sc_mesh(),
        compiler_params=pltpu.CompilerParams(needs_layout_passes=False),
        scratch_types=[
            pltpu.VMEM((SEGR, 128), jnp.float32),
            pltpu.VMEM((EPT,), jnp.int32),
            pltpu.VMEM((EPT,), jnp.float32),
            pltpu.VMEM((5, 128), jnp.int32),
            pltpu.VMEM_SHARED((SEGR, 128), jnp.float32),
            pltpu.SemaphoreType.DMA,
        ],
    )
    return k(seg_e)


def _scatter_body(t2_ref, meta_ref, we_ref, out0_ref, out1_ref,
                  mv_v, wv_v, giA, diA, giB, diB, rowsA, rowsB,
                  acc_sh, gsA, gsB, ssA, ssB):
    c = lax.axis_index("c")
    s = lax.axis_index("s")
    z16 = jnp.zeros((16,), jnp.float32)
    base = pl.multiple_of(s * EPT, 8)

    def idx_chunk(o, gi_ref, di_ref):
        for g in range(5):
            m16 = mv_v[pl.ds(o + g * 16, 16)]
            gi_ref[pl.ds(g * 16, 16)] = lax.shift_right_logical(m16, 14) + c
            di_ref[pl.ds(g * 16, 16)] = m16 & 16383

    def scale(rows_ref, o):
        def sc(g, _):
            w16 = wv_v[pl.ds(o + g * 16, 16)]
            for l in range(16):
                w = w16[l]
                for q in range(8):
                    sl2 = pl.ds(q * 16, 16)
                    rows_ref[g * 16 + l, sl2] = rows_ref[g * 16 + l, sl2] * w
            return 0
        lax.fori_loop(0, 5, sc, 0)

    def gather(gi_ref, rows_ref, sem):
        pltpu.async_copy(t2_ref.at[gi_ref], rows_ref, sem)

    def wait_g(gi_ref, rows_ref, sem):
        pltpu.make_async_copy(t2_ref.at[gi_ref], rows_ref, sem).wait()

    def scatter(rows_ref, di_ref, sem):
        pltpu.async_copy(rows_ref, acc_sh.at[di_ref], sem, add=True)

    def wait_s(rows_ref, di_ref, sem):
        pltpu.make_async_copy(rows_ref, acc_sh.at[di_ref], sem).wait()

    def zb(i, _):
        for g in range(8):
            rowsA[i, pl.ds(g * 16, 16)] = z16
        return 0
    lax.fori_loop(0, CH, zb, 0)
    aoff = pl.multiple_of(s * 640, 8)
    for k in range(8):
        pltpu.sync_copy(rowsA, acc_sh.at[pl.ds(aoff + k * CH, CH)])
    plsc.subcore_barrier()

    for m in range(EPT // MG):
        moff = base + m * MG
        pltpu.sync_copy(meta_ref.at[pl.ds(moff, MG)], mv_v)
        pltpu.sync_copy(we_ref.at[pl.ds(moff, MG)], wv_v)

        idx_chunk(0, giA, diA)
        gather(giA, rowsA, gsA)

        def pair(i, _):
            oa = i * (2 * CH)
            ob = oa + CH
            oa2 = oa + 2 * CH
            @pl.when(i > 0)
            def _():
                wait_s(rowsB, diB, ssB)
            idx_chunk(ob, giB, diB)
            gather(giB, rowsB, gsB)
            wait_g(giA, rowsA, gsA)
            scale(rowsA, oa)
            scatter(rowsA, diA, ssA)
            wait_g(giB, rowsB, gsB)
            scale(rowsB, ob)
            wait_s(rowsA, diA, ssA)
            idx_chunk(oa2, giA, diA)
            gather(giA, rowsA, gsA)
            scatter(rowsB, diB, ssB)
            return 0
        lax.fori_loop(0, (MG // CH - 1) // 2, pair, 0)

        olast = (MG // CH - 1) * CH
        wait_g(giA, rowsA, gsA)
        scale(rowsA, olast)
        pltpu.sync_copy(rowsA, acc_sh.at[diA], add=True)
        wait_s(rowsB, diB, ssB)
    plsc.subcore_barrier()

    @pl.when(jnp.logical_and(c == 0, s < 15))
    def _():
        pltpu.sync_copy(acc_sh.at[pl.ds(aoff, 640)],
                        out0_ref.at[pl.ds(aoff, 640)])

    @pl.when(jnp.logical_and(c == 0, s == 15))
    def _():
        pltpu.sync_copy(acc_sh.at[pl.ds(9600, 400)],
                        out0_ref.at[pl.ds(9600, 400)])

    @pl.when(jnp.logical_and(c == 1, s < 15))
    def _():
        pltpu.sync_copy(acc_sh.at[pl.ds(aoff, 640)],
                        out1_ref.at[pl.ds(aoff, 640)])

    @pl.when(jnp.logical_and(c == 1, s == 15))
    def _():
        pltpu.sync_copy(acc_sh.at[pl.ds(9600, 400)],
                        out1_ref.at[pl.ds(9600, 400)])


def _gather_scatter(trans, meta_e, w_e):
    t2 = trans.reshape(N * R * 2, 128)
    k = pl.kernel(
        _scatter_body,
        out_type=[
            jax.ShapeDtypeStruct((N, 128), jnp.float32),
            jax.ShapeDtypeStruct((N, 128), jnp.float32),
        ],
        mesh=_sc_mesh(),
        compiler_params=pltpu.CompilerParams(needs_layout_passes=False),
        scratch_types=[
            pltpu.VMEM((MG,), jnp.int32),
            pltpu.VMEM((MG,), jnp.float32),
            pltpu.VMEM((CH,), jnp.int32),
            pltpu.VMEM((CH,), jnp.int32),
            pltpu.VMEM((CH,), jnp.int32),
            pltpu.VMEM((CH,), jnp.int32),
            pltpu.VMEM((CH, 128), jnp.float32),
            pltpu.VMEM((CH, 128), jnp.float32),
            pltpu.VMEM_SHARED((10240, 128), jnp.float32),
            pltpu.SemaphoreType.DMA,
            pltpu.SemaphoreType.DMA,
            pltpu.SemaphoreType.DMA,
            pltpu.SemaphoreType.DMA,
        ],
    )
    return k(t2, meta_e, w_e)


def kernel(x_flat, edge_index, edge_type, valid_mask_flat,
           basis_0, comp_0, root_0, bias_0, ln_w_0, ln_b_0,
           basis_1, comp_1, root_1, bias_1, ln_w_1, ln_b_1):
    src_e, dst_e = edge_index[0], edge_index[1]
    seg_e = dst_e * R + edge_type
    gidx2 = src_e * (2 * R) + edge_type * 2
    meta_e = (gidx2 << 14) | dst_e
    w_e = _edge_weights(seg_e)

    h = jnp.where(valid_mask_flat[:, None], x_flat, 0.0)
    w1 = _mix(comp_0, basis_0).astype(jnp.bfloat16)
    w2 = _mix(comp_1, basis_1).astype(jnp.bfloat16)
    rt0 = root_0.astype(jnp.bfloat16)
    rt1 = root_1.astype(jnp.bfloat16)
    trans, hr = _trans(h, w1, rt0)
    a0, a1 = _gather_scatter(trans, meta_e, w_e)
    h1, trans2, hr2 = _post_trans(a0, a1, hr, h, bias_0, ln_w_0, ln_b_0,
                                  w2, rt1)
    b0, b1 = _gather_scatter(trans2, meta_e, w_e)
    h2 = _post(b0, b1, hr2, h1, bias_1, ln_w_1, ln_b_1)
    return jnp.where(valid_mask_flat[:, None], h2, 0.0)


# speedup vs baseline: 4.6867x; 1.0103x over previous
import functools

import jax
import jax.numpy as jnp
from jax import lax
from jax.experimental import pallas as pl
from jax.experimental.pallas import tpu as pltpu
from jax.experimental.pallas import tpu_sc as plsc

N = 10000
D = 256
R = 8
NB = 8
E = 160000
EPS = 1e-5

BN = 1000
SEGR = 640
EPT = E // 16
CH = 80
NCH = EPT // CH
NPT = N // 16
MG = 2000


def _mix_kernel(c0_ref, b0_ref, c1_ref, b1_ref, w0_ref, w1_ref):
    w0_ref[...] = jax.lax.dot_general(
        c0_ref[...], b0_ref[...], (((1,), (0,)), ((), ())),
        preferred_element_type=jnp.float32).astype(jnp.bfloat16)
    w1_ref[...] = jax.lax.dot_general(
        c1_ref[...], b1_ref[...], (((1,), (0,)), ((), ())),
        preferred_element_type=jnp.float32).astype(jnp.bfloat16)


def _mix2(comp0, basis0, comp1, basis1):
    o0, o1 = pl.pallas_call(
        _mix_kernel,
        out_shape=[jax.ShapeDtypeStruct((R, D * D), jnp.bfloat16),
                   jax.ShapeDtypeStruct((R, D * D), jnp.bfloat16)],
    )(comp0, basis0.reshape(NB, D * D), comp1, basis1.reshape(NB, D * D))
    return o0.reshape(R, D, D), o1.reshape(R, D, D)


def _trans_kernel(h_ref, w_ref, root_ref, trans_ref, hr_ref):
    h = h_ref[...].astype(jnp.bfloat16)
    t = jax.lax.dot_general(h, w_ref[...], (((1,), (1,)), ((), ())),
                            preferred_element_type=jnp.float32)
    trans_ref[...] = t.reshape(BN, R * D)
    hr_ref[...] = jnp.dot(h, root_ref[...], preferred_element_type=jnp.float32)


def _trans(h, w, root):
    grid = (N // BN,)
    return pl.pallas_call(
        _trans_kernel,
        grid=grid,
        in_specs=[
            pl.BlockSpec((BN, D), lambda i: (i, 0)),
            pl.BlockSpec((R, D, D), lambda i: (0, 0, 0)),
            pl.BlockSpec((D, D), lambda i: (0, 0)),
        ],
        out_specs=[
            pl.BlockSpec((BN, R * D), lambda i: (i, 0)),
            pl.BlockSpec((BN, D), lambda i: (i, 0)),
        ],
        out_shape=[
            jax.ShapeDtypeStruct((N, R * D), jnp.float32),
            jax.ShapeDtypeStruct((N, D), jnp.float32),
        ],
    )(h, w, root)


def _post_kernel(a0_ref, a1_ref, hr_ref, h_ref, bias_ref, lnw_ref, lnb_ref,
                 mask_ref, out_ref):
    agg = jnp.concatenate([a0_ref[...], a1_ref[...]], axis=1)
    m = agg + hr_ref[...] + bias_ref[...]
    m = 0.5 * m * (1.0 + jax.lax.erf(m * (2.0 ** -0.5)))
    x = h_ref[...] + m
    mu = jnp.mean(x, axis=-1, keepdims=True)
    var = jnp.mean((x - mu) ** 2, axis=-1, keepdims=True)
    y = (x - mu) / jnp.sqrt(var + EPS) * lnw_ref[...] + lnb_ref[...]
    out_ref[...] = y * mask_ref[...]


def _post(a0, a1, hr, h, bias, lnw, lnb, maskf):
    grid = (N // BN,)
    vec = lambda i: (0, 0)
    half = lambda i: (i, 0)
    return pl.pallas_call(
        _post_kernel,
        grid=grid,
        in_specs=[
            pl.BlockSpec((BN, 128), half),
            pl.BlockSpec((BN, 128), half),
            pl.BlockSpec((BN, D), half),
            pl.BlockSpec((BN, D), half),
            pl.BlockSpec((1, D), vec),
            pl.BlockSpec((1, D), vec),
            pl.BlockSpec((1, D), vec),
            pl.BlockSpec((BN, 1), half),
        ],
        out_specs=pl.BlockSpec((BN, D), lambda i: (i, 0)),
        out_shape=jax.ShapeDtypeStruct((N, D), jnp.float32),
    )(a0, a1, hr, h, bias.reshape(1, D), lnw.reshape(1, D), lnb.reshape(1, D),
      maskf)


def _post_trans_kernel(a0_ref, a1_ref, hr_ref, h_ref, bias_ref, lnw_ref,
                       lnb_ref, w_ref, root_ref, hn_ref, trans_ref, hr2_ref):
    agg = jnp.concatenate([a0_ref[...], a1_ref[...]], axis=1)
    m = agg + hr_ref[...] + bias_ref[...]
    m = 0.5 * m * (1.0 + jax.lax.erf(m * (2.0 ** -0.5)))
    x = h_ref[...] + m
    mu = jnp.mean(x, axis=-1, keepdims=True)
    var = jnp.mean((x - mu) ** 2, axis=-1, keepdims=True)
    hn = (x - mu) / jnp.sqrt(var + EPS) * lnw_ref[...] + lnb_ref[...]
    hn_ref[...] = hn
    hb = hn.astype(jnp.bfloat16)
    t = jax.lax.dot_general(hb, w_ref[...], (((1,), (1,)), ((), ())),
                            preferred_element_type=jnp.float32)
    trans_ref[...] = t.reshape(BN, R * D)
    hr2_ref[...] = jnp.dot(hb, root_ref[...],
                           preferred_element_type=jnp.float32)


def _post_trans(a0, a1, hr, h, bias, lnw, lnb, w2, root2):
    grid = (N // BN,)
    vec = lambda i: (0, 0)
    half = lambda i: (i, 0)
    return pl.pallas_call(
        _post_trans_kernel,
        grid=grid,
        in_specs=[
            pl.BlockSpec((BN, 128), half),
            pl.BlockSpec((BN, 128), half),
            pl.BlockSpec((BN, D), half),
            pl.BlockSpec((BN, D), half),
            pl.BlockSpec((1, D), vec),
            pl.BlockSpec((1, D), vec),
            pl.BlockSpec((1, D), vec),
            pl.BlockSpec((R, D, D), lambda i: (0, 0, 0)),
            pl.BlockSpec((D, D), vec),
        ],
        out_specs=[
            pl.BlockSpec((BN, D), half),
            pl.BlockSpec((BN, R * D), half),
            pl.BlockSpec((BN, D), half),
        ],
        out_shape=[
            jax.ShapeDtypeStruct((N, D), jnp.float32),
            jax.ShapeDtypeStruct((N, R * D), jnp.float32),
            jax.ShapeDtypeStruct((N, D), jnp.float32),
        ],
    )(a0, a1, hr, h, bias.reshape(1, D), lnw.reshape(1, D),
      lnb.reshape(1, D), w2, root2)


def _sc_mesh():
    return plsc.VectorSubcoreMesh(core_axis_name="c", subcore_axis_name="s")


def _weights_body(seg_ref, w_ref, hist_v, seg_v, wv_v, ridx_v,
                  cnt_sh, sem):
    c = lax.axis_index("c")
    s = lax.axis_index("s")
    iota = lax.iota(jnp.int32, 16)
    z16 = jnp.zeros((16,), jnp.float32)
    ones = jnp.full((16,), 1.0, jnp.float32)

    @pl.when(c == 0)
    def _():
        def zb(i, _):
            for g in range(8):
                hist_v[i, pl.ds(g * 16, 16)] = z16
            return 0
        lax.fori_loop(0, SEGR, zb, 0)
        zoff = pl.multiple_of(s * 40, 8)
        pltpu.sync_copy(hist_v.at[pl.ds(0, 40)], cnt_sh.at[pl.ds(zoff, 40)])
        for k in range(5):
            for g in range(8):
                ridx_v[k, pl.ds(g * 16, 16)] = iota + (k * 128 + g * 16)
        base = pl.multiple_of(s * EPT, 8)
        pltpu.sync_copy(seg_ref.at[pl.ds(base, EPT)], seg_v)

        def ca(i, _):
            seg = seg_v[pl.ds(i * 16, 16)]
            plsc.addupdate_scatter(hist_v, [seg >> 7, seg & 127], ones)
            return 0
        lax.fori_loop(0, EPT // 16, ca, 0)
        plsc.subcore_barrier()
        for k in range(5):
            pltpu.sync_copy(hist_v.at[pl.ds(k * 128, 128)],
                            cnt_sh.at[ridx_v.at[k]], add=True)
        plsc.subcore_barrier()
        pltpu.sync_copy(cnt_sh, hist_v)

        def cw(i, _):
            seg = seg_v[pl.ds(i * 16, 16)]
            cntv = plsc.load_gather(hist_v, [seg >> 7, seg & 127])
            wv_v[pl.ds(i * 16, 16)] = 1.0 / jnp.maximum(cntv, 1.0)
            return 0
        lax.fori_loop(0, EPT // 16, cw, 0)
        pltpu.sync_copy(wv_v, w_ref.at[pl.ds(base, EPT)])


def _edge_weights(seg_e):
    k = pl.kernel(
        _weights_body,
        out_type=jax.ShapeDtypeStruct((E,), jnp.float32),
        mesh=_sc_mesh(),
        compiler_params=pltpu.CompilerParams(needs_layout_passes=False),
        scratch_types=[
            pltpu.VMEM((SEGR, 128), jnp.float32),
            pltpu.VMEM((EPT,), jnp.int32),
            pltpu.VMEM((EPT,), jnp.float32),
            pltpu.VMEM((5, 128), jnp.int32),
            pltpu.VMEM_SHARED((SEGR, 128), jnp.float32),
            pltpu.SemaphoreType.DMA,
        ],
    )
    return k(seg_e)


def _scatter_body(t2_ref, meta_ref, we_ref, out0_ref, out1_ref,
                  mv_v, wv_v, giA, diA, giB, diB, rowsA, rowsB,
                  acc_sh, gsA, gsB, ssA, ssB):
    c = lax.axis_index("c")
    s = lax.axis_index("s")
    z16 = jnp.zeros((16,), jnp.float32)
    base = pl.multiple_of(s * EPT, 8)

    def idx_chunk(o, gi_ref, di_ref):
        for g in range(5):
            m16 = mv_v[pl.ds(o + g * 16, 16)]
            gi_ref[pl.ds(g * 16, 16)] = lax.shift_right_logical(m16, 14) + c
            di_ref[pl.ds(g * 16, 16)] = m16 & 16383

    def scale(rows_ref, o):
        def sc(g, _):
            w16 = wv_v[pl.ds(o + g * 16, 16)]
            for l in range(16):
                w = w16[l]
                for q in range(8):
                    sl2 = pl.ds(q * 16, 16)
                    rows_ref[g * 16 + l, sl2] = rows_ref[g * 16 + l, sl2] * w
            return 0
        lax.fori_loop(0, 5, sc, 0)

    def gather(gi_ref, rows_ref, sem):
        pltpu.async_copy(t2_ref.at[gi_ref], rows_ref, sem)

    def wait_g(gi_ref, rows_ref, sem):
        pltpu.make_async_copy(t2_ref.at[gi_ref], rows_ref, sem).wait()

    def scatter(rows_ref, di_ref, sem):
        pltpu.async_copy(rows_ref, acc_sh.at[di_ref], sem, add=True)

    def wait_s(rows_ref, di_ref, sem):
        pltpu.make_async_copy(rows_ref, acc_sh.at[di_ref], sem).wait()

    def zb(i, _):
        for g in range(8):
            rowsA[i, pl.ds(g * 16, 16)] = z16
        return 0
    lax.fori_loop(0, CH, zb, 0)
    aoff = pl.multiple_of(s * 640, 8)
    for k in range(8):
        pltpu.sync_copy(rowsA, acc_sh.at[pl.ds(aoff + k * CH, CH)])
    plsc.subcore_barrier()

    for m in range(EPT // MG):
        moff = base + m * MG
        pltpu.sync_copy(meta_ref.at[pl.ds(moff, MG)], mv_v)
        pltpu.sync_copy(we_ref.at[pl.ds(moff, MG)], wv_v)

        idx_chunk(0, giA, diA)
        gather(giA, rowsA, gsA)

        def pair(i, _):
            oa = i * (2 * CH)
            ob = oa + CH
            oa2 = oa + 2 * CH
            @pl.when(i > 0)
            def _():
                wait_s(rowsB, diB, ssB)
            idx_chunk(ob, giB, diB)
            gather(giB, rowsB, gsB)
            wait_g(giA, rowsA, gsA)
            scale(rowsA, oa)
            scatter(rowsA, diA, ssA)
            wait_g(giB, rowsB, gsB)
            scale(rowsB, ob)
            wait_s(rowsA, diA, ssA)
            idx_chunk(oa2, giA, diA)
            gather(giA, rowsA, gsA)
            scatter(rowsB, diB, ssB)
            return 0
        lax.fori_loop(0, (MG // CH - 1) // 2, pair, 0)

        olast = (MG // CH - 1) * CH
        wait_g(giA, rowsA, gsA)
        scale(rowsA, olast)
        pltpu.sync_copy(rowsA, acc_sh.at[diA], add=True)
        wait_s(rowsB, diB, ssB)
    plsc.subcore_barrier()

    @pl.when(jnp.logical_and(c == 0, s < 15))
    def _():
        pltpu.sync_copy(acc_sh.at[pl.ds(aoff, 640)],
                        out0_ref.at[pl.ds(aoff, 640)])

    @pl.when(jnp.logical_and(c == 0, s == 15))
    def _():
        pltpu.sync_copy(acc_sh.at[pl.ds(9600, 400)],
                        out0_ref.at[pl.ds(9600, 400)])

    @pl.when(jnp.logical_and(c == 1, s < 15))
    def _():
        pltpu.sync_copy(acc_sh.at[pl.ds(aoff, 640)],
                        out1_ref.at[pl.ds(aoff, 640)])

    @pl.when(jnp.logical_and(c == 1, s == 15))
    def _():
        pltpu.sync_copy(acc_sh.at[pl.ds(9600, 400)],
                        out1_ref.at[pl.ds(9600, 400)])


def _gather_scatter(trans, meta_e, w_e):
    t2 = trans.reshape(N * R * 2, 128)
    k = pl.kernel(
        _scatter_body,
        out_type=[
            jax.ShapeDtypeStruct((N, 128), jnp.float32),
            jax.ShapeDtypeStruct((N, 128), jnp.float32),
        ],
        mesh=_sc_mesh(),
        compiler_params=pltpu.CompilerParams(needs_layout_passes=False),
        scratch_types=[
            pltpu.VMEM((MG,), jnp.int32),
            pltpu.VMEM((MG,), jnp.float32),
            pltpu.VMEM((CH,), jnp.int32),
            pltpu.VMEM((CH,), jnp.int32),
            pltpu.VMEM((CH,), jnp.int32),
            pltpu.VMEM((CH,), jnp.int32),
            pltpu.VMEM((CH, 128), jnp.float32),
            pltpu.VMEM((CH, 128), jnp.float32),
            pltpu.VMEM_SHARED((10240, 128), jnp.float32),
            pltpu.SemaphoreType.DMA,
            pltpu.SemaphoreType.DMA,
            pltpu.SemaphoreType.DMA,
            pltpu.SemaphoreType.DMA,
        ],
    )
    return k(t2, meta_e, w_e)


def kernel(x_flat, edge_index, edge_type, valid_mask_flat,
           basis_0, comp_0, root_0, bias_0, ln_w_0, ln_b_0,
           basis_1, comp_1, root_1, bias_1, ln_w_1, ln_b_1):
    src_e, dst_e = edge_index[0], edge_index[1]
    seg_e = dst_e * R + edge_type
    gidx2 = src_e * (2 * R) + edge_type * 2
    meta_e = (gidx2 << 14) | dst_e
    w_e = _edge_weights(seg_e)

    h = jnp.where(valid_mask_flat[:, None], x_flat, 0.0)
    w1, w2 = _mix2(comp_0, basis_0, comp_1, basis_1)
    rt0 = root_0.astype(jnp.bfloat16)
    rt1 = root_1.astype(jnp.bfloat16)
    trans, hr = _trans(h, w1, rt0)
    a0, a1 = _gather_scatter(trans, meta_e, w_e)
    h1, trans2, hr2 = _post_trans(a0, a1, hr, h, bias_0, ln_w_0, ln_b_0,
                                  w2, rt1)
    b0, b1 = _gather_scatter(trans2, meta_e, w_e)
    maskf = valid_mask_flat.astype(jnp.float32).reshape(N, 1)
    return _post(b0, b1, hr2, h1, bias_1, ln_w_1, ln_b_1, maskf)


# speedup vs baseline: 4.8830x; 1.0419x over previous
import functools

import jax
import jax.numpy as jnp
from jax import lax
from jax.experimental import pallas as pl
from jax.experimental.pallas import tpu as pltpu
from jax.experimental.pallas import tpu_sc as plsc

N = 10000
D = 256
R = 8
NB = 8
E = 160000
EPS = 1e-5

BN = 1000
SEGR = 640
EPT = E // 16
CH = 80
NCH = EPT // CH
NPT = N // 16
MG = 10000


def _mix_kernel(c0_ref, b0_ref, c1_ref, b1_ref, w0_ref, w1_ref):
    w0_ref[...] = jax.lax.dot_general(
        c0_ref[...], b0_ref[...], (((1,), (0,)), ((), ())),
        preferred_element_type=jnp.float32).astype(jnp.bfloat16)
    w1_ref[...] = jax.lax.dot_general(
        c1_ref[...], b1_ref[...], (((1,), (0,)), ((), ())),
        preferred_element_type=jnp.float32).astype(jnp.bfloat16)


def _mix2(comp0, basis0, comp1, basis1):
    o0, o1 = pl.pallas_call(
        _mix_kernel,
        out_shape=[jax.ShapeDtypeStruct((R, D * D), jnp.bfloat16),
                   jax.ShapeDtypeStruct((R, D * D), jnp.bfloat16)],
    )(comp0, basis0.reshape(NB, D * D), comp1, basis1.reshape(NB, D * D))
    return o0.reshape(R, D, D), o1.reshape(R, D, D)


def _trans_kernel(h_ref, w_ref, root_ref, trans_ref, hr_ref):
    h = h_ref[...].astype(jnp.bfloat16)
    t = jax.lax.dot_general(h, w_ref[...], (((1,), (1,)), ((), ())),
                            preferred_element_type=jnp.float32)
    trans_ref[...] = t.reshape(BN, R * D)
    hr_ref[...] = jnp.dot(h, root_ref[...], preferred_element_type=jnp.float32)


def _trans(h, w, root):
    grid = (N // BN,)
    return pl.pallas_call(
        _trans_kernel,
        grid=grid,
        in_specs=[
            pl.BlockSpec((BN, D), lambda i: (i, 0)),
            pl.BlockSpec((R, D, D), lambda i: (0, 0, 0)),
            pl.BlockSpec((D, D), lambda i: (0, 0)),
        ],
        out_specs=[
            pl.BlockSpec((BN, R * D), lambda i: (i, 0)),
            pl.BlockSpec((BN, D), lambda i: (i, 0)),
        ],
        out_shape=[
            jax.ShapeDtypeStruct((N, R * D), jnp.float32),
            jax.ShapeDtypeStruct((N, D), jnp.float32),
        ],
    )(h, w, root)


def _post_kernel(a0_ref, a1_ref, hr_ref, h_ref, bias_ref, lnw_ref, lnb_ref,
                 mask_ref, out_ref):
    agg = jnp.concatenate([a0_ref[...], a1_ref[...]], axis=1)
    m = agg + hr_ref[...] + bias_ref[...]
    m = 0.5 * m * (1.0 + jax.lax.erf(m * (2.0 ** -0.5)))
    x = h_ref[...] + m
    mu = jnp.mean(x, axis=-1, keepdims=True)
    var = jnp.mean((x - mu) ** 2, axis=-1, keepdims=True)
    y = (x - mu) / jnp.sqrt(var + EPS) * lnw_ref[...] + lnb_ref[...]
    out_ref[...] = y * mask_ref[...]


def _post(a0, a1, hr, h, bias, lnw, lnb, maskf):
    grid = (N // BN,)
    vec = lambda i: (0, 0)
    half = lambda i: (i, 0)
    return pl.pallas_call(
        _post_kernel,
        grid=grid,
        in_specs=[
            pl.BlockSpec((BN, 128), half),
            pl.BlockSpec((BN, 128), half),
            pl.BlockSpec((BN, D), half),
            pl.BlockSpec((BN, D), half),
            pl.BlockSpec((1, D), vec),
            pl.BlockSpec((1, D), vec),
            pl.BlockSpec((1, D), vec),
            pl.BlockSpec((BN, 1), half),
        ],
        out_specs=pl.BlockSpec((BN, D), lambda i: (i, 0)),
        out_shape=jax.ShapeDtypeStruct((N, D), jnp.float32),
    )(a0, a1, hr, h, bias.reshape(1, D), lnw.reshape(1, D), lnb.reshape(1, D),
      maskf)


def _post_trans_kernel(a0_ref, a1_ref, hr_ref, h_ref, bias_ref, lnw_ref,
                       lnb_ref, w_ref, root_ref, hn_ref, trans_ref, hr2_ref):
    agg = jnp.concatenate([a0_ref[...], a1_ref[...]], axis=1)
    m = agg + hr_ref[...] + bias_ref[...]
    m = 0.5 * m * (1.0 + jax.lax.erf(m * (2.0 ** -0.5)))
    x = h_ref[...] + m
    mu = jnp.mean(x, axis=-1, keepdims=True)
    var = jnp.mean((x - mu) ** 2, axis=-1, keepdims=True)
    hn = (x - mu) / jnp.sqrt(var + EPS) * lnw_ref[...] + lnb_ref[...]
    hn_ref[...] = hn
    hb = hn.astype(jnp.bfloat16)
    t = jax.lax.dot_general(hb, w_ref[...], (((1,), (1,)), ((), ())),
                            preferred_element_type=jnp.float32)
    trans_ref[...] = t.reshape(BN, R * D)
    hr2_ref[...] = jnp.dot(hb, root_ref[...],
                           preferred_element_type=jnp.float32)


def _post_trans(a0, a1, hr, h, bias, lnw, lnb, w2, root2):
    grid = (N // BN,)
    vec = lambda i: (0, 0)
    half = lambda i: (i, 0)
    return pl.pallas_call(
        _post_trans_kernel,
        grid=grid,
        in_specs=[
            pl.BlockSpec((BN, 128), half),
            pl.BlockSpec((BN, 128), half),
            pl.BlockSpec((BN, D), half),
            pl.BlockSpec((BN, D), half),
            pl.BlockSpec((1, D), vec),
            pl.BlockSpec((1, D), vec),
            pl.BlockSpec((1, D), vec),
            pl.BlockSpec((R, D, D), lambda i: (0, 0, 0)),
            pl.BlockSpec((D, D), vec),
        ],
        out_specs=[
            pl.BlockSpec((BN, D), half),
            pl.BlockSpec((BN, R * D), half),
            pl.BlockSpec((BN, D), half),
        ],
        out_shape=[
            jax.ShapeDtypeStruct((N, D), jnp.float32),
            jax.ShapeDtypeStruct((N, R * D), jnp.float32),
            jax.ShapeDtypeStruct((N, D), jnp.float32),
        ],
    )(a0, a1, hr, h, bias.reshape(1, D), lnw.reshape(1, D),
      lnb.reshape(1, D), w2, root2)


def _sc_mesh():
    return plsc.VectorSubcoreMesh(core_axis_name="c", subcore_axis_name="s")


def _weights_body(seg_ref, w_ref, hist_v, seg_v, wv_v, ridx_v,
                  cnt_sh, sem):
    c = lax.axis_index("c")
    s = lax.axis_index("s")
    iota = lax.iota(jnp.int32, 16)
    z16 = jnp.zeros((16,), jnp.float32)
    ones = jnp.full((16,), 1.0, jnp.float32)

    @pl.when(c == 0)
    def _():
        def zb(i, _):
            for g in range(8):
                hist_v[i, pl.ds(g * 16, 16)] = z16
            return 0
        lax.fori_loop(0, SEGR, zb, 0)
        zoff = pl.multiple_of(s * 40, 8)
        pltpu.sync_copy(hist_v.at[pl.ds(0, 40)], cnt_sh.at[pl.ds(zoff, 40)])
        for k in range(5):
            for g in range(8):
                ridx_v[k, pl.ds(g * 16, 16)] = iota + (k * 128 + g * 16)
        base = pl.multiple_of(s * EPT, 8)
        pltpu.sync_copy(seg_ref.at[pl.ds(base, EPT)], seg_v)

        def ca(i, _):
            seg = seg_v[pl.ds(i * 16, 16)]
            plsc.addupdate_scatter(hist_v, [seg >> 7, seg & 127], ones)
            return 0
        lax.fori_loop(0, EPT // 16, ca, 0)
        plsc.subcore_barrier()
        for k in range(5):
            pltpu.sync_copy(hist_v.at[pl.ds(k * 128, 128)],
                            cnt_sh.at[ridx_v.at[k]], add=True)
        plsc.subcore_barrier()
        pltpu.sync_copy(cnt_sh, hist_v)

        def cw(i, _):
            seg = seg_v[pl.ds(i * 16, 16)]
            cntv = plsc.load_gather(hist_v, [seg >> 7, seg & 127])
            wv_v[pl.ds(i * 16, 16)] = 1.0 / jnp.maximum(cntv, 1.0)
            return 0
        lax.fori_loop(0, EPT // 16, cw, 0)
        pltpu.sync_copy(wv_v, w_ref.at[pl.ds(base, EPT)])


def _edge_weights(seg_e):
    k = pl.kernel(
        _weights_body,
        out_type=jax.ShapeDtypeStruct((E,), jnp.float32),
        mesh=_sc_mesh(),
        compiler_params=pltpu.CompilerParams(needs_layout_passes=False),
        scratch_types=[
            pltpu.VMEM((SEGR, 128), jnp.float32),
            pltpu.VMEM((EPT,), jnp.int32),
            pltpu.VMEM((EPT,), jnp.float32),
            pltpu.VMEM((5, 128), jnp.int32),
            pltpu.VMEM_SHARED((SEGR, 128), jnp.float32),
            pltpu.SemaphoreType.DMA,
        ],
    )
    return k(seg_e)


def _scatter_body(t2_ref, meta_ref, we_ref, out0_ref, out1_ref,
                  mv_v, wv_v, giA, diA, giB, diB, rowsA, rowsB,
                  acc_sh, gsA, gsB, ssA, ssB):
    c = lax.axis_index("c")
    s = lax.axis_index("s")
    z16 = jnp.zeros((16,), jnp.float32)
    base = pl.multiple_of(s * EPT, 8)

    def idx_chunk(o, gi_ref, di_ref):
        for g in range(5):
            m16 = mv_v[pl.ds(o + g * 16, 16)]
            gi_ref[pl.ds(g * 16, 16)] = lax.shift_right_logical(m16, 14) + c
            di_ref[pl.ds(g * 16, 16)] = m16 & 16383

    def scale(rows_ref, o):
        def sc(g, _):
            w16 = wv_v[pl.ds(o + g * 16, 16)]
            for l in range(16):
                w = w16[l]
                for q in range(8):
                    sl2 = pl.ds(q * 16, 16)
                    rows_ref[g * 16 + l, sl2] = rows_ref[g * 16 + l, sl2] * w
            return 0
        lax.fori_loop(0, 5, sc, 0)

    def gather(gi_ref, rows_ref, sem):
        pltpu.async_copy(t2_ref.at[gi_ref], rows_ref, sem)

    def wait_g(gi_ref, rows_ref, sem):
        pltpu.make_async_copy(t2_ref.at[gi_ref], rows_ref, sem).wait()

    def scatter(rows_ref, di_ref, sem):
        pltpu.async_copy(rows_ref, acc_sh.at[di_ref], sem, add=True)

    def wait_s(rows_ref, di_ref, sem):
        pltpu.make_async_copy(rows_ref, acc_sh.at[di_ref], sem).wait()

    def zb(i, _):
        for g in range(8):
            rowsA[i, pl.ds(g * 16, 16)] = z16
        return 0
    lax.fori_loop(0, CH, zb, 0)
    aoff = pl.multiple_of(s * 640, 8)

    @pl.when(s < 15)
    def _():
        for k in range(8):
            pltpu.sync_copy(rowsA, acc_sh.at[pl.ds(aoff + k * CH, CH)])

    @pl.when(s == 15)
    def _():
        for k in range(5):
            pltpu.sync_copy(rowsA, acc_sh.at[pl.ds(9600 + k * CH, CH)])
    plsc.subcore_barrier()

    for m in range(EPT // MG):
        moff = base + m * MG
        pltpu.sync_copy(meta_ref.at[pl.ds(moff, MG)], mv_v)
        pltpu.sync_copy(we_ref.at[pl.ds(moff, MG)], wv_v)

        idx_chunk(0, giA, diA)
        gather(giA, rowsA, gsA)

        def pair(i, _):
            oa = i * (2 * CH)
            ob = oa + CH
            oa2 = oa + 2 * CH
            @pl.when(i > 0)
            def _():
                wait_s(rowsB, diB, ssB)
            idx_chunk(ob, giB, diB)
            gather(giB, rowsB, gsB)
            wait_g(giA, rowsA, gsA)
            scale(rowsA, oa)
            scatter(rowsA, diA, ssA)
            wait_g(giB, rowsB, gsB)
            scale(rowsB, ob)
            wait_s(rowsA, diA, ssA)
            idx_chunk(oa2, giA, diA)
            gather(giA, rowsA, gsA)
            scatter(rowsB, diB, ssB)
            return 0
        lax.fori_loop(0, (MG // CH - 1) // 2, pair, 0)

        olast = (MG // CH - 1) * CH
        wait_g(giA, rowsA, gsA)
        scale(rowsA, olast)
        pltpu.sync_copy(rowsA, acc_sh.at[diA], add=True)
        wait_s(rowsB, diB, ssB)
    plsc.subcore_barrier()

    @pl.when(jnp.logical_and(c == 0, s < 15))
    def _():
        pltpu.sync_copy(acc_sh.at[pl.ds(aoff, 640)],
                        out0_ref.at[pl.ds(aoff, 640)])

    @pl.when(jnp.logical_and(c == 0, s == 15))
    def _():
        pltpu.sync_copy(acc_sh.at[pl.ds(9600, 400)],
                        out0_ref.at[pl.ds(9600, 400)])

    @pl.when(jnp.logical_and(c == 1, s < 15))
    def _():
        pltpu.sync_copy(acc_sh.at[pl.ds(aoff, 640)],
                        out1_ref.at[pl.ds(aoff, 640)])

    @pl.when(jnp.logical_and(c == 1, s == 15))
    def _():
        pltpu.sync_copy(acc_sh.at[pl.ds(9600, 400)],
                        out1_ref.at[pl.ds(9600, 400)])


def _gather_scatter(trans, meta_e, w_e):
    t2 = trans.reshape(N * R * 2, 128)
    k = pl.kernel(
        _scatter_body,
        out_type=[
            jax.ShapeDtypeStruct((N, 128), jnp.float32),
            jax.ShapeDtypeStruct((N, 128), jnp.float32),
        ],
        mesh=_sc_mesh(),
        compiler_params=pltpu.CompilerParams(needs_layout_passes=False),
        scratch_types=[
            pltpu.VMEM((MG,), jnp.int32),
            pltpu.VMEM((MG,), jnp.float32),
            pltpu.VMEM((CH,), jnp.int32),
            pltpu.VMEM((CH,), jnp.int32),
            pltpu.VMEM((CH,), jnp.int32),
            pltpu.VMEM((CH,), jnp.int32),
            pltpu.VMEM((CH, 128), jnp.float32),
            pltpu.VMEM((CH, 128), jnp.float32),
            pltpu.VMEM_SHARED((N, 128), jnp.float32),
            pltpu.SemaphoreType.DMA,
            pltpu.SemaphoreType.DMA,
            pltpu.SemaphoreType.DMA,
            pltpu.SemaphoreType.DMA,
        ],
    )
    return k(t2, meta_e, w_e)


def kernel(x_flat, edge_index, edge_type, valid_mask_flat,
           basis_0, comp_0, root_0, bias_0, ln_w_0, ln_b_0,
           basis_1, comp_1, root_1, bias_1, ln_w_1, ln_b_1):
    src_e, dst_e = edge_index[0], edge_index[1]
    seg_e = dst_e * R + edge_type
    gidx2 = src_e * (2 * R) + edge_type * 2
    meta_e = (gidx2 << 14) | dst_e
    w_e = _edge_weights(seg_e)

    h = jnp.where(valid_mask_flat[:, None], x_flat, 0.0)
    w1, w2 = _mix2(comp_0, basis_0, comp_1, basis_1)
    rt0 = root_0.astype(jnp.bfloat16)
    rt1 = root_1.astype(jnp.bfloat16)
    trans, hr = _trans(h, w1, rt0)
    a0, a1 = _gather_scatter(trans, meta_e, w_e)
    h1, trans2, hr2 = _post_trans(a0, a1, hr, h, bias_0, ln_w_0, ln_b_0,
                                  w2, rt1)
    b0, b1 = _gather_scatter(trans2, meta_e, w_e)
    maskf = valid_mask_flat.astype(jnp.float32).reshape(N, 1)
    return _post(b0, b1, hr2, h1, bias_1, ln_w_1, ln_b_1, maskf)


# speedup vs baseline: 4.8951x; 1.0025x over previous
import jax
import jax.numpy as jnp
from jax import lax
from jax.experimental import pallas as pl
from jax.experimental.pallas import tpu as pltpu
from jax.experimental.pallas import tpu_sc as plsc

N = 10000
D = 256
R = 8
NB = 8
E = 160000
EPS = 1e-5

BN = 1000
SEGR = 640
EPT = E // 16
CH = 80
MG = 10000


def _mix_kernel(c0_ref, b0_ref, c1_ref, b1_ref, w0_ref, w1_ref):
    w0_ref[...] = jax.lax.dot_general(
        c0_ref[...], b0_ref[...], (((1,), (0,)), ((), ())),
        preferred_element_type=jnp.float32).astype(jnp.bfloat16)
    w1_ref[...] = jax.lax.dot_general(
        c1_ref[...], b1_ref[...], (((1,), (0,)), ((), ())),
        preferred_element_type=jnp.float32).astype(jnp.bfloat16)


def _mix2(comp0, basis0, comp1, basis1):
    o0, o1 = pl.pallas_call(
        _mix_kernel,
        out_shape=[jax.ShapeDtypeStruct((R, D * D), jnp.bfloat16),
                   jax.ShapeDtypeStruct((R, D * D), jnp.bfloat16)],
    )(comp0, basis0.reshape(NB, D * D), comp1, basis1.reshape(NB, D * D))
    return o0.reshape(R, D, D), o1.reshape(R, D, D)


def _trans_kernel(h_ref, w_ref, root_ref, trans_ref, hr_ref):
    h = h_ref[...].astype(jnp.bfloat16)
    t = jax.lax.dot_general(h, w_ref[...], (((1,), (1,)), ((), ())),
                            preferred_element_type=jnp.float32)
    trans_ref[...] = t.reshape(BN, R * D)
    hr_ref[...] = jnp.dot(h, root_ref[...], preferred_element_type=jnp.float32)


def _trans(h, w, root):
    grid = (N // BN,)
    return pl.pallas_call(
        _trans_kernel,
        grid=grid,
        in_specs=[
            pl.BlockSpec((BN, D), lambda i: (i, 0)),
            pl.BlockSpec((R, D, D), lambda i: (0, 0, 0)),
            pl.BlockSpec((D, D), lambda i: (0, 0)),
        ],
        out_specs=[
            pl.BlockSpec((BN, R * D), lambda i: (i, 0)),
            pl.BlockSpec((BN, D), lambda i: (i, 0)),
        ],
        out_shape=[
            jax.ShapeDtypeStruct((N, R * D), jnp.float32),
            jax.ShapeDtypeStruct((N, D), jnp.float32),
        ],
    )(h, w, root)


def _post_kernel(a0_ref, a1_ref, hr_ref, h_ref, bias_ref, lnw_ref, lnb_ref,
                 mask_ref, out_ref):
    agg = jnp.concatenate([a0_ref[...], a1_ref[...]], axis=1)
    m = agg + hr_ref[...] + bias_ref[...]
    m = 0.5 * m * (1.0 + jax.lax.erf(m * (2.0 ** -0.5)))
    x = h_ref[...] + m
    mu = jnp.mean(x, axis=-1, keepdims=True)
    var = jnp.mean((x - mu) ** 2, axis=-1, keepdims=True)
    y = (x - mu) / jnp.sqrt(var + EPS) * lnw_ref[...] + lnb_ref[...]
    out_ref[...] = y * mask_ref[...]


def _post(a0, a1, hr, h, bias, lnw, lnb, maskf):
    grid = (N // BN,)
    vec = lambda i: (0, 0)
    half = lambda i: (i, 0)
    return pl.pallas_call(
        _post_kernel,
        grid=grid,
        in_specs=[
            pl.BlockSpec((BN, 128), half),
            pl.BlockSpec((BN, 128), half),
            pl.BlockSpec((BN, D), half),
            pl.BlockSpec((BN, D), half),
            pl.BlockSpec((1, D), vec),
            pl.BlockSpec((1, D), vec),
            pl.BlockSpec((1, D), vec),
            pl.BlockSpec((BN, 1), half),
        ],
        out_specs=pl.BlockSpec((BN, D), lambda i: (i, 0)),
        out_shape=jax.ShapeDtypeStruct((N, D), jnp.float32),
    )(a0, a1, hr, h, bias.reshape(1, D), lnw.reshape(1, D), lnb.reshape(1, D),
      maskf)


def _post_trans_kernel(a0_ref, a1_ref, hr_ref, h_ref, bias_ref, lnw_ref,
                       lnb_ref, w_ref, root_ref, hn_ref, trans_ref, hr2_ref):
    agg = jnp.concatenate([a0_ref[...], a1_ref[...]], axis=1)
    m = agg + hr_ref[...] + bias_ref[...]
    m = 0.5 * m * (1.0 + jax.lax.erf(m * (2.0 ** -0.5)))
    x = h_ref[...] + m
    mu = jnp.mean(x, axis=-1, keepdims=True)
    var = jnp.mean((x - mu) ** 2, axis=-1, keepdims=True)
    hn = (x - mu) / jnp.sqrt(var + EPS) * lnw_ref[...] + lnb_ref[...]
    hn_ref[...] = hn
    hb = hn.astype(jnp.bfloat16)
    t = jax.lax.dot_general(hb, w_ref[...], (((1,), (1,)), ((), ())),
                            preferred_element_type=jnp.float32)
    trans_ref[...] = t.reshape(BN, R * D)
    hr2_ref[...] = jnp.dot(hb, root_ref[...],
                           preferred_element_type=jnp.float32)


def _post_trans(a0, a1, hr, h, bias, lnw, lnb, w2, root2):
    grid = (N // BN,)
    vec = lambda i: (0, 0)
    half = lambda i: (i, 0)
    return pl.pallas_call(
        _post_trans_kernel,
        grid=grid,
        in_specs=[
            pl.BlockSpec((BN, 128), half),
            pl.BlockSpec((BN, 128), half),
            pl.BlockSpec((BN, D), half),
            pl.BlockSpec((BN, D), half),
            pl.BlockSpec((1, D), vec),
            pl.BlockSpec((1, D), vec),
            pl.BlockSpec((1, D), vec),
            pl.BlockSpec((R, D, D), lambda i: (0, 0, 0)),
            pl.BlockSpec((D, D), vec),
        ],
        out_specs=[
            pl.BlockSpec((BN, D), half),
            pl.BlockSpec((BN, R * D), half),
            pl.BlockSpec((BN, D), half),
        ],
        out_shape=[
            jax.ShapeDtypeStruct((N, D), jnp.float32),
            jax.ShapeDtypeStruct((N, R * D), jnp.float32),
            jax.ShapeDtypeStruct((N, D), jnp.float32),
        ],
    )(a0, a1, hr, h, bias.reshape(1, D), lnw.reshape(1, D),
      lnb.reshape(1, D), w2, root2)


def _sc_mesh():
    return plsc.VectorSubcoreMesh(core_axis_name="c", subcore_axis_name="s")


def _weights_body(seg_ref, w_ref, hist_v, seg_v, wv_v, ridx_v,
                  cnt_sh, sem):
    c = lax.axis_index("c")
    s = lax.axis_index("s")
    iota = lax.iota(jnp.int32, 16)
    z16 = jnp.zeros((16,), jnp.float32)
    ones = jnp.full((16,), 1.0, jnp.float32)

    @pl.when(c == 0)
    def _():
        def zb(i, _):
            for g in range(8):
                hist_v[i, pl.ds(g * 16, 16)] = z16
            return 0
        lax.fori_loop(0, SEGR, zb, 0)
        zoff = pl.multiple_of(s * 40, 8)
        pltpu.sync_copy(hist_v.at[pl.ds(0, 40)], cnt_sh.at[pl.ds(zoff, 40)])
        for k in range(5):
            for g in range(8):
                ridx_v[k, pl.ds(g * 16, 16)] = iota + (k * 128 + g * 16)
        base = pl.multiple_of(s * EPT, 8)
        pltpu.sync_copy(seg_ref.at[pl.ds(base, EPT)], seg_v)

        def ca(i, _):
            seg = seg_v[pl.ds(i * 16, 16)]
            plsc.addupdate_scatter(hist_v, [seg >> 7, seg & 127], ones)
            return 0
        lax.fori_loop(0, EPT // 16, ca, 0)
        plsc.subcore_barrier()
        for k in range(5):
            pltpu.sync_copy(hist_v.at[pl.ds(k * 128, 128)],
                            cnt_sh.at[ridx_v.at[k]], add=True)
        plsc.subcore_barrier()
        pltpu.sync_copy(cnt_sh, hist_v)

        def cw(i, _):
            seg = seg_v[pl.ds(i * 16, 16)]
            cntv = plsc.load_gather(hist_v, [seg >> 7, seg & 127])
            wv_v[pl.ds(i * 16, 16)] = 1.0 / jnp.maximum(cntv, 1.0)
            return 0
        lax.fori_loop(0, EPT // 16, cw, 0)
        pltpu.sync_copy(wv_v, w_ref.at[pl.ds(base, EPT)])


def _edge_weights(seg_e):
    k = pl.kernel(
        _weights_body,
        out_type=jax.ShapeDtypeStruct((E,), jnp.float32),
        mesh=_sc_mesh(),
        compiler_params=pltpu.CompilerParams(needs_layout_passes=False),
        scratch_types=[
            pltpu.VMEM((SEGR, 128), jnp.float32),
            pltpu.VMEM((EPT,), jnp.int32),
            pltpu.VMEM((EPT,), jnp.float32),
            pltpu.VMEM((5, 128), jnp.int32),
            pltpu.VMEM_SHARED((SEGR, 128), jnp.float32),
            pltpu.SemaphoreType.DMA,
        ],
    )
    return k(seg_e)


def _scatter_body(t2_ref, meta_ref, we_ref, out0_ref, out1_ref,
                  mv_v, wv_v, giA, diA, giB, diB, rowsA, rowsB,
                  acc_sh, gsA, gsB, ssA, ssB):
    c = lax.axis_index("c")
    s = lax.axis_index("s")
    z16 = jnp.zeros((16,), jnp.float32)
    base = pl.multiple_of(s * EPT, 8)

    def idx_chunk(o, gi_ref, di_ref):
        for g in range(5):
            m16 = mv_v[pl.ds(o + g * 16, 16)]
            gi_ref[pl.ds(g * 16, 16)] = lax.shift_right_logical(m16, 14) + c
            di_ref[pl.ds(g * 16, 16)] = m16 & 16383

    def scale(rows_ref, o):
        def sc(g, _):
            w16 = wv_v[pl.ds(o + g * 16, 16)]
            for l in range(16):
                w = w16[l]
                for q in range(8):
                    sl2 = pl.ds(q * 16, 16)
                    rows_ref[g * 16 + l, sl2] = rows_ref[g * 16 + l, sl2] * w
            return 0
        lax.fori_loop(0, 5, sc, 0)

    def gather(gi_ref, rows_ref, sem):
        pltpu.async_copy(t2_ref.at[gi_ref], rows_ref, sem)

    def wait_g(gi_ref, rows_ref, sem):
        pltpu.make_async_copy(t2_ref.at[gi_ref], rows_ref, sem).wait()

    def scatter(rows_ref, di_ref, sem):
        pltpu.async_copy(rows_ref, acc_sh.at[di_ref], sem, add=True)

    def wait_s(rows_ref, di_ref, sem):
        pltpu.make_async_copy(rows_ref, acc_sh.at[di_ref], sem).wait()

    def zb(i, _):
        for g in range(8):
            rowsA[i, pl.ds(g * 16, 16)] = z16
        return 0
    lax.fori_loop(0, CH, zb, 0)
    aoff = pl.multiple_of(s * 640, 8)

    @pl.when(s < 15)
    def _():
        for k in range(8):
            pltpu.sync_copy(rowsA, acc_sh.at[pl.ds(aoff + k * CH, CH)])

    @pl.when(s == 15)
    def _():
        for k in range(5):
            pltpu.sync_copy(rowsA, acc_sh.at[pl.ds(9600 + k * CH, CH)])
    plsc.subcore_barrier()

    for m in range(EPT // MG):
        moff = base + m * MG
        pltpu.sync_copy(meta_ref.at[pl.ds(moff, MG)], mv_v)
        pltpu.sync_copy(we_ref.at[pl.ds(moff, MG)], wv_v)

        idx_chunk(0, giA, diA)
        gather(giA, rowsA, gsA)

        def pair(i, _):
            oa = i * (2 * CH)
            ob = oa + CH
            oa2 = oa + 2 * CH
            @pl.when(i > 0)
            def _():
                wait_s(rowsB, diB, ssB)
            idx_chunk(ob, giB, diB)
            gather(giB, rowsB, gsB)
            wait_g(giA, rowsA, gsA)
            scale(rowsA, oa)
            scatter(rowsA, diA, ssA)
            wait_g(giB, rowsB, gsB)
            scale(rowsB, ob)
            wait_s(rowsA, diA, ssA)
            idx_chunk(oa2, giA, diA)
            gather(giA, rowsA, gsA)
            scatter(rowsB, diB, ssB)
            return 0
        lax.fori_loop(0, (MG // CH - 1) // 2, pair, 0)

        olast = (MG // CH - 1) * CH
        wait_g(giA, rowsA, gsA)
        scale(rowsA, olast)
        pltpu.sync_copy(rowsA, acc_sh.at[diA], add=True)
        wait_s(rowsB, diB, ssB)
    plsc.subcore_barrier()

    @pl.when(jnp.logical_and(c == 0, s < 15))
    def _():
        pltpu.sync_copy(acc_sh.at[pl.ds(aoff, 640)],
                        out0_ref.at[pl.ds(aoff, 640)])

    @pl.when(jnp.logical_and(c == 0, s == 15))
    def _():
        pltpu.sync_copy(acc_sh.at[pl.ds(9600, 400)],
                        out0_ref.at[pl.ds(9600, 400)])

    @pl.when(jnp.logical_and(c == 1, s < 15))
    def _():
        pltpu.sync_copy(acc_sh.at[pl.ds(aoff, 640)],
                        out1_ref.at[pl.ds(aoff, 640)])

    @pl.when(jnp.logical_and(c == 1, s == 15))
    def _():
        pltpu.sync_copy(acc_sh.at[pl.ds(9600, 400)],
                        out1_ref.at[pl.ds(9600, 400)])


def _gather_scatter(trans, meta_e, w_e):
    t2 = trans.reshape(N * R * 2, 128)
    k = pl.kernel(
        _scatter_body,
        out_type=[
            jax.ShapeDtypeStruct((N, 128), jnp.float32),
            jax.ShapeDtypeStruct((N, 128), jnp.float32),
        ],
        mesh=_sc_mesh(),
        compiler_params=pltpu.CompilerParams(needs_layout_passes=False),
        scratch_types=[
            pltpu.VMEM((MG,), jnp.int32),
            pltpu.VMEM((MG,), jnp.float32),
            pltpu.VMEM((CH,), jnp.int32),
            pltpu.VMEM((CH,), jnp.int32),
            pltpu.VMEM((CH,), jnp.int32),
            pltpu.VMEM((CH,), jnp.int32),
            pltpu.VMEM((CH, 128), jnp.float32),
            pltpu.VMEM((CH, 128), jnp.float32),
            pltpu.VMEM_SHARED((N, 128), jnp.float32),
            pltpu.SemaphoreType.DMA,
            pltpu.SemaphoreType.DMA,
            pltpu.SemaphoreType.DMA,
            pltpu.SemaphoreType.DMA,
        ],
    )
    return k(t2, meta_e, w_e)


def kernel(x_flat, edge_index, edge_type, valid_mask_flat,
           basis_0, comp_0, root_0, bias_0, ln_w_0, ln_b_0,
           basis_1, comp_1, root_1, bias_1, ln_w_1, ln_b_1):
    src_e, dst_e = edge_index[0], edge_index[1]
    seg_e = dst_e * R + edge_type
    gidx2 = src_e * (2 * R) + edge_type * 2
    meta_e = (gidx2 << 14) | dst_e
    w_e = _edge_weights(seg_e)

    h = jnp.where(valid_mask_flat[:, None], x_flat, 0.0)
    w1, w2 = _mix2(comp_0, basis_0, comp_1, basis_1)
    rt0 = root_0.astype(jnp.bfloat16)
    rt1 = root_1.astype(jnp.bfloat16)
    trans, hr = _trans(h, w1, rt0)
    a0, a1 = _gather_scatter(trans, meta_e, w_e)
    h1, trans2, hr2 = _post_trans(a0, a1, hr, h, bias_0, ln_w_0, ln_b_0,
                                  w2, rt1)
    b0, b1 = _gather_scatter(trans2, meta_e, w_e)
    maskf = valid_mask_flat.astype(jnp.float32).reshape(N, 1)
    return _post(b0, b1, hr2, h1, bias_1, ln_w_1, ln_b_1, maskf)
